# baseline (speedup 1.0000x reference)
import functools

import jax
import jax.numpy as jnp
from jax import lax
from jax.experimental import pallas as pl
from jax.experimental.pallas import tpu as pltpu
from jax.experimental.pallas import tpu_sc as plsc

N = 10000
E = 320000
D = 128
NP = 10240

NCORE = 2
NSUB = 16
CHUNK = 128
CPT = 157
EPAD = NSUB * CPT * CHUNK
ZONE = NP // NCORE


def _fill(ref, n, val):
    v = jnp.full((16,), val, ref.dtype)

    def body(i, _):
        ref[pl.ds(i * 16, 16)] = v
        return 0

    lax.fori_loop(0, n // 16, body, 0)


ZBUF = 32768


def _adj_body(src_hbm, dst_hbm, out_hbm, zbuf, srcv, dstv, idxv, onesv):
    c = lax.axis_index("c")
    s = lax.axis_index("s")
    zelem = ZONE * NP // NSUB
    zbase = c * (ZONE * NP) + s * zelem
    _fill(zbuf, ZBUF, 0.0)

    def zcopy(i, _):
        pltpu.sync_copy(zbuf, out_hbm.at[pl.ds(zbase + i * ZBUF, ZBUF)])
        return 0

    lax.fori_loop(0, zelem // ZBUF, zcopy, 0)
    plsc.subcore_barrier()

    _fill(onesv, CHUNK, 1.0)
    trash = (c * ZONE) * NP + c * ZONE
    zlo = c * ZONE
    zhi = zlo + ZONE

    def chunk_body(j, _):
        off = (s * CPT + j) * CHUNK
        pltpu.sync_copy(src_hbm.at[pl.ds(off, CHUNK)], srcv)
        pltpu.sync_copy(dst_hbm.at[pl.ds(off, CHUNK)], dstv)
        for v in range(CHUNK // 16):
            sl = pl.ds(v * 16, 16)
            sv = srcv[sl]
            dv = dstv[sl]
            fl = dv * NP + sv
            inz = (dv >= zlo) & (dv < zhi)
            idxv[sl] = jnp.where(inz, fl, trash)
        pltpu.sync_copy(onesv, out_hbm.at[idxv])
        return 0

    lax.fori_loop(0, CPT, chunk_body, 0)


def _adjacency_sc(src_pad, dst_pad):
    mesh = plsc.VectorSubcoreMesh(core_axis_name="c", subcore_axis_name="s")
    f = pl.kernel(
        _adj_body,
        out_type=jax.ShapeDtypeStruct((NP * NP,), jnp.float32),
        mesh=mesh,
        scratch_types=[
            pltpu.VMEM((ZBUF,), jnp.float32),
            pltpu.VMEM((CHUNK,), jnp.int32),
            pltpu.VMEM((CHUNK,), jnp.int32),
            pltpu.VMEM((CHUNK,), jnp.int32),
            pltpu.VMEM((CHUNK,), jnp.float32),
        ],
    )
    return f(src_pad, dst_pad)

EGO_BI = 1024
EGO_BK = 256


def _ego_body(a_ref, b_ref, x_ref, o_ref, cnt_ref):
    k = pl.program_id(1)
    nk = pl.num_programs(1)
    counts = jnp.dot(a_ref[...], b_ref[...], preferred_element_type=jnp.float32)
    direct = a_ref[:, pl.ds(k * EGO_BK, EGO_BK)].astype(jnp.float32)
    tot = counts + direct
    i = pl.program_id(0)
    rows = i * EGO_BI + lax.broadcasted_iota(jnp.int32, (EGO_BI, EGO_BK), 0)
    cols = k * EGO_BK + lax.broadcasted_iota(jnp.int32, (EGO_BI, EGO_BK), 1)
    reach = jnp.where((tot > 0.5) | (rows == cols), 1.0, 0.0)
    contrib = jnp.dot(reach, x_ref[...], preferred_element_type=jnp.float32)
    rc = jnp.sum(reach, axis=1, keepdims=True)

    @pl.when(k == 0)
    def _():
        cnt_ref[...] = rc
        o_ref[...] = contrib

    @pl.when(k > 0)
    def _():
        cnt_ref[...] += rc
        o_ref[...] += contrib

    @pl.when(k == nk - 1)
    def _():
        o_ref[...] = o_ref[...] / cnt_ref[...]


def _ego_pallas(Bbf, x_pad):
    grid = (NP // EGO_BI, NP // EGO_BK)
    return pl.pallas_call(
        _ego_body,
        grid=grid,
        in_specs=[
            pl.BlockSpec((EGO_BI, NP), lambda i, k: (i, 0)),
            pl.BlockSpec((NP, EGO_BK), lambda i, k: (0, k)),
            pl.BlockSpec((EGO_BK, D), lambda i, k: (k, 0)),
        ],
        out_specs=pl.BlockSpec((EGO_BI, D), lambda i, k: (i, 0)),
        out_shape=jax.ShapeDtypeStruct((NP, D), jnp.float32),
        scratch_shapes=[pltpu.VMEM((EGO_BI, 1), jnp.float32)],
    )(Bbf, Bbf, x_pad)


def _mid_body(x_ref, ego_ref, cutn_ref, cutd_ref, cosn_ref, cosd_ref,
              we_ref, be_ref, wc_ref, bc_ref, wco_ref, bco_ref, wg_ref, bg_ref,
              h_ref, glob_ref):
    x = x_ref[...]
    den = cutd_ref[...]
    has_nb = den > 0.0
    cut = jnp.where(has_nb, cutn_ref[...] / jnp.maximum(den, 1e-12), x)
    cosd = cosd_ref[...]
    cosine = jnp.where(has_nb, cosn_ref[...] / jnp.maximum(cosd, 1e-12), x)
    ego = ego_ref[...]
    he = jnp.dot(ego, we_ref[...].T, preferred_element_type=jnp.float32) + be_ref[...]
    hc = jnp.dot(cut, wc_ref[...].T, preferred_element_type=jnp.float32) + bc_ref[...]
    ho = jnp.dot(cosine, wco_ref[...].T, preferred_element_type=jnp.float32) + bco_ref[...]
    h_ref[...] = jnp.concatenate([he, hc, ho], axis=1)
    glob_ref[...] = jnp.dot(x, wg_ref[...].T, preferred_element_type=jnp.float32) + bg_ref[...]


def _mid_pallas(x, ego, cut_num, cut_den, cos_num, cos_den,
                W_ego, b_ego, W_cut, b_cut, W_cos, b_cos, W_glob, b_glob):
    BR = 1000
    grid = (N // BR,)
    row = pl.BlockSpec((BR, D), lambda i: (i, 0))
    rowc = pl.BlockSpec((BR, 1), lambda i: (i, 0))
    wspec = pl.BlockSpec((D, D), lambda i: (0, 0))
    bspec = pl.BlockSpec((1, D), lambda i: (0, 0))
    return pl.pallas_call(
        _mid_body,
        grid=grid,
        in_specs=[row, row, row, rowc, row, rowc,
                  wspec, bspec, wspec, bspec, wspec, bspec, wspec, bspec],
        out_specs=[pl.BlockSpec((BR, 3 * D), lambda i: (i, 0)), row],
        out_shape=[jax.ShapeDtypeStruct((N, 3 * D), jnp.float32),
                   jax.ShapeDtypeStruct((N, D), jnp.float32)],
    )(x, ego, cut_num, cut_den, cos_num, cos_den,
      W_ego, b_ego.reshape(1, D), W_cut, b_cut.reshape(1, D),
      W_cos, b_cos.reshape(1, D), W_glob, b_glob.reshape(1, D))


def _tail_body(agg_ref, glob_ref, wfc_ref, bfc_ref, o_ref):
    comb = jnp.concatenate([jax.nn.relu(agg_ref[...]), glob_ref[...]], axis=1)
    logits = jnp.dot(comb, wfc_ref[...].T, preferred_element_type=jnp.float32) + bfc_ref[...]
    m = jnp.max(logits, axis=1, keepdims=True)
    s = logits - m
    lse = jnp.log(jnp.sum(jnp.exp(s), axis=1, keepdims=True))
    o_ref[...] = s - lse


def _tail_pallas(agg, glob, W_fc, b_fc):
    BR = 1000
    grid = (N // BR,)
    return pl.pallas_call(
        _tail_body,
        grid=grid,
        in_specs=[
            pl.BlockSpec((BR, 3 * D), lambda i: (i, 0)),
            pl.BlockSpec((BR, D), lambda i: (i, 0)),
            pl.BlockSpec((D, 4 * D), lambda i: (0, 0)),
            pl.BlockSpec((1, D), lambda i: (0, 0)),
        ],
        out_specs=pl.BlockSpec((BR, D), lambda i: (i, 0)),
        out_shape=jax.ShapeDtypeStruct((N, D), jnp.float32),
    )(agg, glob, W_fc, b_fc.reshape(1, D))


def kernel(x, edge_index, W_ego, b_ego, W_cut, b_cut, W_cos, b_cos,
           W_glob, b_glob, W_fc, b_fc):
    src = edge_index[0]
    dst = edge_index[1]

    pad = jnp.full((EPAD - E,), N, jnp.int32)
    src_pad = jnp.concatenate([src, pad])
    dst_pad = jnp.concatenate([dst, pad])
    B01 = _adjacency_sc(src_pad, dst_pad).reshape(NP, NP)
    Bbf = B01.astype(jnp.bfloat16)
    x_pad = jnp.pad(x, ((0, NP - N), (0, 0)))

    ego = _ego_pallas(Bbf, x_pad)[:N]

    ones = jnp.ones((E,), jnp.float32)
    cut_num = jax.ops.segment_sum(x[dst], src, num_segments=N)
    cut_den = jax.ops.segment_sum(ones, src, num_segments=N)

    nx = x / jnp.maximum(jnp.linalg.norm(x, axis=1, keepdims=True), 1e-12)
    cos = jnp.sum(nx[dst] * nx[src], axis=1)
    e = jnp.exp(cos)
    s = jax.ops.segment_sum(e, src, num_segments=N)
    cos_num = jax.ops.segment_sum(x[dst] * e[:, None], src, num_segments=N)

    h_all, glob = _mid_pallas(
        x, ego, cut_num, cut_den.reshape(N, 1), cos_num, s.reshape(N, 1),
        W_ego, b_ego, W_cut, b_cut, W_cos, b_cos, W_glob, b_glob)

    agg = jax.ops.segment_sum(h_all[src], dst, num_segments=N)

    return _tail_pallas(agg, glob, W_fc, b_fc)


# speedup vs baseline: 3.7201x; 3.7201x over previous
import functools

import jax
import jax.numpy as jnp
from jax import lax
from jax.experimental import pallas as pl
from jax.experimental.pallas import tpu as pltpu
from jax.experimental.pallas import tpu_sc as plsc

N = 10000
E = 320000
D = 128
NP = 10240

NCORE = 2
NSUB = 16
CHUNK = 128
CPT = 157
EPAD = NSUB * CPT * CHUNK
ZONE = NP // NCORE


def _fill(ref, n, val):
    v = jnp.full((16,), val, ref.dtype)

    def body(i, _):
        ref[pl.ds(i * 16, 16)] = v
        return 0

    lax.fori_loop(0, n // 16, body, 0)


ZBUF = 32768


def _adj_body(src_hbm, dst_hbm, out_hbm, zbuf, srcv, dstv, idxv, onesv):
    c = lax.axis_index("c")
    s = lax.axis_index("s")
    zelem = ZONE * NP // NSUB
    zbase = c * (ZONE * NP) + s * zelem
    _fill(zbuf, ZBUF, 0.0)

    def zcopy(i, _):
        pltpu.sync_copy(zbuf, out_hbm.at[pl.ds(zbase + i * ZBUF, ZBUF)])
        return 0

    lax.fori_loop(0, zelem // ZBUF, zcopy, 0)
    plsc.subcore_barrier()

    _fill(onesv, CHUNK, 1.0)
    zlo = c * ZONE
    zhi = zlo + ZONE

    def chunk_body(j, _):
        off = (s * CPT + j) * CHUNK
        pltpu.sync_copy(src_hbm.at[pl.ds(off, CHUNK)], srcv)
        pltpu.sync_copy(dst_hbm.at[pl.ds(off, CHUNK)], dstv)
        for v in range(CHUNK // 16):
            sl = pl.ds(v * 16, 16)
            sv = srcv[sl]
            dv = dstv[sl]
            fl = dv * NP + sv
            inz = (dv >= zlo) & (dv < zhi)
            trow = zlo + (sv & 4095)
            idxv[sl] = jnp.where(inz, fl, trow * (NP + 1))
        pltpu.sync_copy(onesv, out_hbm.at[idxv])
        return 0

    lax.fori_loop(0, CPT, chunk_body, 0)


def _adjacency_sc(src_pad, dst_pad):
    mesh = plsc.VectorSubcoreMesh(core_axis_name="c", subcore_axis_name="s")
    f = pl.kernel(
        _adj_body,
        out_type=jax.ShapeDtypeStruct((NP * NP,), jnp.float32),
        mesh=mesh,
        scratch_types=[
            pltpu.VMEM((ZBUF,), jnp.float32),
            pltpu.VMEM((CHUNK,), jnp.int32),
            pltpu.VMEM((CHUNK,), jnp.int32),
            pltpu.VMEM((CHUNK,), jnp.int32),
            pltpu.VMEM((CHUNK,), jnp.float32),
        ],
    )
    return f(src_pad, dst_pad)

EGO_BI = 1024
EGO_BK = 256


def _ego_body(a_ref, b_ref, x_ref, o_ref, cnt_ref):
    k = pl.program_id(1)
    nk = pl.num_programs(1)
    counts = jnp.dot(a_ref[...], b_ref[...], preferred_element_type=jnp.float32)
    direct = a_ref[:, pl.ds(k * EGO_BK, EGO_BK)].astype(jnp.float32)
    tot = counts + direct
    i = pl.program_id(0)
    rows = i * EGO_BI + lax.broadcasted_iota(jnp.int32, (EGO_BI, EGO_BK), 0)
    cols = k * EGO_BK + lax.broadcasted_iota(jnp.int32, (EGO_BI, EGO_BK), 1)
    reach = jnp.where((tot > 0.5) | (rows == cols), 1.0, 0.0)
    contrib = jnp.dot(reach, x_ref[...], preferred_element_type=jnp.float32)
    rc = jnp.sum(reach, axis=1, keepdims=True)

    @pl.when(k == 0)
    def _():
        cnt_ref[...] = rc
        o_ref[...] = contrib

    @pl.when(k > 0)
    def _():
        cnt_ref[...] += rc
        o_ref[...] += contrib

    @pl.when(k == nk - 1)
    def _():
        o_ref[...] = o_ref[...] / cnt_ref[...]


def _ego_pallas(Bbf, x_pad):
    grid = (NP // EGO_BI, NP // EGO_BK)
    return pl.pallas_call(
        _ego_body,
        grid=grid,
        in_specs=[
            pl.BlockSpec((EGO_BI, NP), lambda i, k: (i, 0)),
            pl.BlockSpec((NP, EGO_BK), lambda i, k: (0, k)),
            pl.BlockSpec((EGO_BK, D), lambda i, k: (k, 0)),
        ],
        out_specs=pl.BlockSpec((EGO_BI, D), lambda i, k: (i, 0)),
        out_shape=jax.ShapeDtypeStruct((NP, D), jnp.float32),
        scratch_shapes=[pltpu.VMEM((EGO_BI, 1), jnp.float32)],
    )(Bbf, Bbf, x_pad)


def _mid_body(x_ref, ego_ref, cutn_ref, cutd_ref, cosn_ref, cosd_ref,
              we_ref, be_ref, wc_ref, bc_ref, wco_ref, bco_ref, wg_ref, bg_ref,
              h_ref, glob_ref):
    x = x_ref[...]
    den = cutd_ref[...]
    has_nb = den > 0.0
    cut = jnp.where(has_nb, cutn_ref[...] / jnp.maximum(den, 1e-12), x)
    cosd = cosd_ref[...]
    cosine = jnp.where(has_nb, cosn_ref[...] / jnp.maximum(cosd, 1e-12), x)
    ego = ego_ref[...]
    he = jnp.dot(ego, we_ref[...].T, preferred_element_type=jnp.float32) + be_ref[...]
    hc = jnp.dot(cut, wc_ref[...].T, preferred_element_type=jnp.float32) + bc_ref[...]
    ho = jnp.dot(cosine, wco_ref[...].T, preferred_element_type=jnp.float32) + bco_ref[...]
    h_ref[...] = jnp.concatenate([he, hc, ho], axis=1)
    glob_ref[...] = jnp.dot(x, wg_ref[...].T, preferred_element_type=jnp.float32) + bg_ref[...]


def _mid_pallas(x, ego, cut_num, cut_den, cos_num, cos_den,
                W_ego, b_ego, W_cut, b_cut, W_cos, b_cos, W_glob, b_glob):
    BR = 1000
    grid = (N // BR,)
    row = pl.BlockSpec((BR, D), lambda i: (i, 0))
    rowc = pl.BlockSpec((BR, 1), lambda i: (i, 0))
    wspec = pl.BlockSpec((D, D), lambda i: (0, 0))
    bspec = pl.BlockSpec((1, D), lambda i: (0, 0))
    return pl.pallas_call(
        _mid_body,
        grid=grid,
        in_specs=[row, row, row, rowc, row, rowc,
                  wspec, bspec, wspec, bspec, wspec, bspec, wspec, bspec],
        out_specs=[pl.BlockSpec((BR, 3 * D), lambda i: (i, 0)), row],
        out_shape=[jax.ShapeDtypeStruct((N, 3 * D), jnp.float32),
                   jax.ShapeDtypeStruct((N, D), jnp.float32)],
    )(x, ego, cut_num, cut_den, cos_num, cos_den,
      W_ego, b_ego.reshape(1, D), W_cut, b_cut.reshape(1, D),
      W_cos, b_cos.reshape(1, D), W_glob, b_glob.reshape(1, D))


def _tail_body(agg_ref, glob_ref, wfc_ref, bfc_ref, o_ref):
    comb = jnp.concatenate([jax.nn.relu(agg_ref[...]), glob_ref[...]], axis=1)
    logits = jnp.dot(comb, wfc_ref[...].T, preferred_element_type=jnp.float32) + bfc_ref[...]
    m = jnp.max(logits, axis=1, keepdims=True)
    s = logits - m
    lse = jnp.log(jnp.sum(jnp.exp(s), axis=1, keepdims=True))
    o_ref[...] = s - lse


def _tail_pallas(agg, glob, W_fc, b_fc):
    BR = 1000
    grid = (N // BR,)
    return pl.pallas_call(
        _tail_body,
        grid=grid,
        in_specs=[
            pl.BlockSpec((BR, 3 * D), lambda i: (i, 0)),
            pl.BlockSpec((BR, D), lambda i: (i, 0)),
            pl.BlockSpec((D, 4 * D), lambda i: (0, 0)),
            pl.BlockSpec((1, D), lambda i: (0, 0)),
        ],
        out_specs=pl.BlockSpec((BR, D), lambda i: (i, 0)),
        out_shape=jax.ShapeDtypeStruct((N, D), jnp.float32),
    )(agg, glob, W_fc, b_fc.reshape(1, D))


def kernel(x, edge_index, W_ego, b_ego, W_cut, b_cut, W_cos, b_cos,
           W_glob, b_glob, W_fc, b_fc):
    src = edge_index[0]
    dst = edge_index[1]

    pad_src = N + (jnp.arange(EPAD - E, dtype=jnp.int32) % 16)
    pad_dst = jnp.full((EPAD - E,), N, jnp.int32)
    src_pad = jnp.concatenate([src, pad_src])
    dst_pad = jnp.concatenate([dst, pad_dst])
    B01 = _adjacency_sc(src_pad, dst_pad).reshape(NP, NP)
    Bbf = B01.astype(jnp.bfloat16)
    x_pad = jnp.pad(x, ((0, NP - N), (0, 0)))

    ego = _ego_pallas(Bbf, x_pad)[:N]

    ones = jnp.ones((E,), jnp.float32)
    cut_num = jax.ops.segment_sum(x[dst], src, num_segments=N)
    cut_den = jax.ops.segment_sum(ones, src, num_segments=N)

    nx = x / jnp.maximum(jnp.linalg.norm(x, axis=1, keepdims=True), 1e-12)
    cos = jnp.sum(nx[dst] * nx[src], axis=1)
    e = jnp.exp(cos)
    s = jax.ops.segment_sum(e, src, num_segments=N)
    cos_num = jax.ops.segment_sum(x[dst] * e[:, None], src, num_segments=N)

    h_all, glob = _mid_pallas(
        x, ego, cut_num, cut_den.reshape(N, 1), cos_num, s.reshape(N, 1),
        W_ego, b_ego, W_cut, b_cut, W_cos, b_cos, W_glob, b_glob)

    agg = jax.ops.segment_sum(h_all[src], dst, num_segments=N)

    return _tail_pallas(agg, glob, W_fc, b_fc)


# speedup vs baseline: 6.2577x; 1.6821x over previous
import jax
import jax.numpy as jnp
from jax import lax
from jax.experimental import pallas as pl
from jax.experimental.pallas import tpu as pltpu
from jax.experimental.pallas import tpu_sc as plsc

N = 10000
E = 320000
D = 128
NP = 10240

NCORE = 2
NSUB = 16
CHUNK = 128
CPT = 157
EPAD = NSUB * CPT * CHUNK
NCHUNK = EPAD // CHUNK
ZONE = NP // NCORE

NT = NP
ZPT = NT // NSUB


def _fill(ref, n, val):
    v = jnp.full((16,), val, ref.dtype)

    def body(i, _):
        ref[pl.ds(i * 16, 16)] = v
        return 0

    lax.fori_loop(0, n // 16, body, 0)


def _zero2d(ref, nrow, cols):
    nv = cols // 16

    def body(i, _):
        r = i // nv
        cblk = (i % nv) * 16
        ref[r, pl.ds(cblk, 16)] = jnp.zeros((16,), jnp.float32)
        return 0

    lax.fori_loop(0, nrow * nv, body, 0)


ZBUF = 32768


def _adj_body(src_hbm, dst_hbm, out_hbm, zbuf, srcv, dstv, idxv, onesv):
    c = lax.axis_index("c")
    s = lax.axis_index("s")
    zelem = ZONE * NP // NSUB
    zbase = c * (ZONE * NP) + s * zelem
    _fill(zbuf, ZBUF, 0.0)

    def zcopy(i, _):
        pltpu.sync_copy(zbuf, out_hbm.at[pl.ds(zbase + i * ZBUF, ZBUF)])
        return 0

    lax.fori_loop(0, zelem // ZBUF, zcopy, 0)
    plsc.subcore_barrier()

    _fill(onesv, CHUNK, 1.0)
    zlo = c * ZONE
    zhi = zlo + ZONE

    def chunk_body(j, _):
        off = (s * CPT + j) * CHUNK
        pltpu.sync_copy(src_hbm.at[pl.ds(off, CHUNK)], srcv)
        pltpu.sync_copy(dst_hbm.at[pl.ds(off, CHUNK)], dstv)
        for v in range(CHUNK // 16):
            sl = pl.ds(v * 16, 16)
            sv = srcv[sl]
            dv = dstv[sl]
            fl = dv * NP + sv
            inz = (dv >= zlo) & (dv < zhi)
            trow = zlo + (sv & 4095)
            idxv[sl] = jnp.where(inz, fl, trow * (NP + 1))
        pltpu.sync_copy(onesv, out_hbm.at[idxv])
        return 0

    lax.fori_loop(0, CPT, chunk_body, 0)


def _adjacency_sc(src_pad, dst_pad):
    mesh = plsc.VectorSubcoreMesh(core_axis_name="c", subcore_axis_name="s")
    f = pl.kernel(
        _adj_body,
        out_type=jax.ShapeDtypeStruct((NP * NP,), jnp.float32),
        mesh=mesh,
        scratch_types=[
            pltpu.VMEM((ZBUF,), jnp.float32),
            pltpu.VMEM((CHUNK,), jnp.int32),
            pltpu.VMEM((CHUNK,), jnp.int32),
            pltpu.VMEM((CHUNK,), jnp.int32),
            pltpu.VMEM((CHUNK,), jnp.float32),
        ],
    )
    return f(src_pad, dst_pad)


def _cutcos_body(src_hbm, dst_hbm, cflat_hbm, x_hbm,
                 cutn_hbm, cosn_hbm, den_hbm, sume_hbm,
                 acc_t, acc1_t, buf2d, z1d,
                 srcv, dstv, idxv, ev, onesv, rows):
    c = lax.axis_index("c")
    s = lax.axis_index("s")

    _zero2d(buf2d, CHUNK, D)
    for kk in range(ZPT // CHUNK):
        pltpu.sync_copy(buf2d, acc_t.at[pl.ds(s * ZPT + kk * CHUNK, CHUNK), :])

    @pl.when(s == 0)
    def _():
        _fill(z1d, 2048, 0.0)
        for kk in range(NT // 2048):
            pltpu.sync_copy(z1d, acc1_t.at[pl.ds(kk * 2048, 2048)])

    plsc.subcore_barrier()
    _fill(onesv, CHUNK, 1.0)

    def chunk_cut(j, _):
        off = (s * CPT + j) * CHUNK
        pltpu.sync_copy(src_hbm.at[pl.ds(off, CHUNK)], srcv)
        pltpu.sync_copy(dst_hbm.at[pl.ds(off, CHUNK)], dstv)
        pltpu.sync_copy(x_hbm.at[dstv], rows)
        pltpu.sync_copy(rows, acc_t.at[srcv], add=True)
        pltpu.sync_copy(onesv, acc1_t.at[srcv], add=True)
        return 0

    def chunk_cos(j, _):
        off = (s * CPT + j) * CHUNK
        pltpu.sync_copy(src_hbm.at[pl.ds(off, CHUNK)], srcv)
        pltpu.sync_copy(dst_hbm.at[pl.ds(off, CHUNK)], dstv)
        for v in range(CHUNK // 16):
            sl = pl.ds(v * 16, 16)
            idxv[sl] = srcv[sl] * NP + dstv[sl]
        pltpu.sync_copy(cflat_hbm.at[idxv], ev)
        for v in range(CHUNK // 16):
            sl = pl.ds(v * 16, 16)
            ev[sl] = jnp.exp(ev[sl])
        pltpu.sync_copy(x_hbm.at[dstv], rows)

        def scale(g, _):
            evec = ev[pl.ds(g * 16, 16)]
            for rr in range(16):
                eb = evec.at[jnp.full((16,), rr, jnp.int32)].get(
                    mode="promise_in_bounds")
                r = g * 16 + rr
                for q in range(D // 16):
                    qs = pl.ds(q * 16, 16)
                    rows[r, qs] = rows[r, qs] * eb
            return 0

        lax.fori_loop(0, CHUNK // 16, scale, 0)
        pltpu.sync_copy(rows, acc_t.at[srcv], add=True)
        pltpu.sync_copy(ev, acc1_t.at[srcv], add=True)
        return 0

    @pl.when(c == 0)
    def _():
        lax.fori_loop(0, CPT, chunk_cut, 0)

    @pl.when(c == 1)
    def _():
        lax.fori_loop(0, CPT, chunk_cos, 0)

    plsc.subcore_barrier()

    r0 = s * ZPT
    for kk in range(ZPT // CHUNK):
        rr = pl.ds(r0 + kk * CHUNK, CHUNK)
        pltpu.sync_copy(acc_t.at[rr, :], buf2d)

        @pl.when(c == 0)
        def _():
            pltpu.sync_copy(buf2d, cutn_hbm.at[rr, :])

        @pl.when(c == 1)
        def _():
            pltpu.sync_copy(buf2d, cosn_hbm.at[rr, :])

    @pl.when(s == 0)
    def _():
        for kk in range(NT // 2048):
            r2 = pl.ds(kk * 2048, 2048)
            pltpu.sync_copy(acc1_t.at[r2], z1d)

            @pl.when(c == 0)
            def _():
                pltpu.sync_copy(z1d, den_hbm.at[r2])

            @pl.when(c == 1)
            def _():
                pltpu.sync_copy(z1d, sume_hbm.at[r2])


def _cutcos_sc(src_pad, dst_pad, cflat, x_pad):
    mesh = plsc.VectorSubcoreMesh(core_axis_name="c", subcore_axis_name="s")
    f = pl.kernel(
        _cutcos_body,
        out_type=[
            jax.ShapeDtypeStruct((NT, D), jnp.float32),
            jax.ShapeDtypeStruct((NT, D), jnp.float32),
            jax.ShapeDtypeStruct((NT,), jnp.float32),
            jax.ShapeDtypeStruct((NT,), jnp.float32),
        ],
        mesh=mesh,
        scratch_types=[
            pltpu.VMEM_SHARED((NT, D), jnp.float32),
            pltpu.VMEM_SHARED((NT,), jnp.float32),
            pltpu.VMEM((CHUNK, D), jnp.float32),
            pltpu.VMEM((2048,), jnp.float32),
            pltpu.VMEM((CHUNK,), jnp.int32),
            pltpu.VMEM((CHUNK,), jnp.int32),
            pltpu.VMEM((CHUNK,), jnp.int32),
            pltpu.VMEM((CHUNK,), jnp.float32),
            pltpu.VMEM((CHUNK,), jnp.float32),
            pltpu.VMEM((CHUNK, D), jnp.float32),
        ],
    )
    return f(src_pad, dst_pad, cflat, x_pad)


def _mpagg_body(src_hbm, dst_hbm, he_hbm, hc_hbm, ho_hbm,
                agge_hbm, aggc_hbm, aggo_hbm,
                acc_t, buf2d, srcv, dstv, rows):
    c = lax.axis_index("c")
    s = lax.axis_index("s")

    def zero_acc():
        _zero2d(buf2d, CHUNK, D)
        for kk in range(ZPT // CHUNK):
            pltpu.sync_copy(
                buf2d, acc_t.at[pl.ds(s * ZPT + kk * CHUNK, CHUNK), :])

    def accumulate(h_hbm, off):
        pltpu.sync_copy(src_hbm.at[pl.ds(off, CHUNK)], srcv)
        pltpu.sync_copy(dst_hbm.at[pl.ds(off, CHUNK)], dstv)
        pltpu.sync_copy(h_hbm.at[srcv], rows)
        pltpu.sync_copy(rows, acc_t.at[dstv], add=True)

    def copy_out(dst_hbm_plane):
        r0 = s * ZPT
        for kk in range(ZPT // CHUNK):
            rr = pl.ds(r0 + kk * CHUNK, CHUNK)
            pltpu.sync_copy(acc_t.at[rr, :], buf2d)
            pltpu.sync_copy(buf2d, dst_hbm_plane.at[rr, :])

    zero_acc()
    plsc.subcore_barrier()

    def p1_e(j, _):
        accumulate(he_hbm, (s * CPT + j) * CHUNK)
        return 0

    def p1_c(j, _):
        accumulate(hc_hbm, (s * CPT + j) * CHUNK)
        return 0

    @pl.when(c == 0)
    def _():
        lax.fori_loop(0, CPT, p1_e, 0)

    @pl.when(c == 1)
    def _():
        lax.fori_loop(0, CPT, p1_c, 0)

    plsc.subcore_barrier()

    @pl.when(c == 0)
    def _():
        copy_out(agge_hbm)

    @pl.when(c == 1)
    def _():
        copy_out(aggc_hbm)

    plsc.subcore_barrier()
    zero_acc()
    plsc.subcore_barrier()

    half_chunks = NCHUNK // 2

    def p2(j, _):
        cid = s + j * NSUB

        @pl.when(cid < half_chunks)
        def _():
            accumulate(ho_hbm, (c * half_chunks + cid) * CHUNK)

        return 0

    lax.fori_loop(0, (half_chunks + NSUB - 1) // NSUB, p2, 0)
    plsc.subcore_barrier()
    copy_out(aggo_hbm.at[c])


def _mpagg_sc(src_pad, dst_pad, he, hc, ho):
    mesh = plsc.VectorSubcoreMesh(core_axis_name="c", subcore_axis_name="s")
    f = pl.kernel(
        _mpagg_body,
        out_type=[
            jax.ShapeDtypeStruct((NT, D), jnp.float32),
            jax.ShapeDtypeStruct((NT, D), jnp.float32),
            jax.ShapeDtypeStruct((NCORE, NT, D), jnp.float32),
        ],
        mesh=mesh,
        scratch_types=[
            pltpu.VMEM_SHARED((NT, D), jnp.float32),
            pltpu.VMEM((CHUNK, D), jnp.float32),
            pltpu.VMEM((CHUNK,), jnp.int32),
            pltpu.VMEM((CHUNK,), jnp.int32),
            pltpu.VMEM((CHUNK, D), jnp.float32),
        ],
    )
    return f(src_pad, dst_pad, he, hc, ho)


CB_I = 1000
CB_J = 1024


def _ctab_body(xi_ref, xj_ref, o_ref):
    xi = xi_ref[...]
    ni = xi / jnp.maximum(
        jnp.sqrt(jnp.sum(xi * xi, axis=1, keepdims=True)), 1e-12)
    xj = xj_ref[...]
    nj = xj / jnp.maximum(
        jnp.sqrt(jnp.sum(xj * xj, axis=1, keepdims=True)), 1e-12)
    o_ref[...] = lax.dot_general(ni, nj, (((1,), (1,)), ((), ())),
                                 preferred_element_type=jnp.float32)


def _ctab_pallas(x, x_pad):
    grid = (N // CB_I, NP // CB_J)
    return pl.pallas_call(
        _ctab_body,
        grid=grid,
        in_specs=[
            pl.BlockSpec((CB_I, D), lambda i, j: (i, 0)),
            pl.BlockSpec((CB_J, D), lambda i, j: (j, 0)),
        ],
        out_specs=pl.BlockSpec((CB_I, CB_J), lambda i, j: (i, j)),
        out_shape=jax.ShapeDtypeStruct((N, NP), jnp.float32),
    )(x, x_pad)


EGO_BI = 1024
EGO_BK = 256


def _ego_body(a_ref, b_ref, x_ref, o_ref, cnt_ref):
    k = pl.program_id(1)
    nk = pl.num_programs(1)
    counts = jnp.dot(a_ref[...], b_ref[...], preferred_element_type=jnp.float32)
    direct = a_ref[:, pl.ds(k * EGO_BK, EGO_BK)].astype(jnp.float32)
    tot = counts + direct
    i = pl.program_id(0)
    rows = i * EGO_BI + lax.broadcasted_iota(jnp.int32, (EGO_BI, EGO_BK), 0)
    cols = k * EGO_BK + lax.broadcasted_iota(jnp.int32, (EGO_BI, EGO_BK), 1)
    reach = jnp.where((tot > 0.5) | (rows == cols), 1.0, 0.0)
    contrib = jnp.dot(reach, x_ref[...], preferred_element_type=jnp.float32)
    rc = jnp.sum(reach, axis=1, keepdims=True)

    @pl.when(k == 0)
    def _():
        cnt_ref[...] = rc
        o_ref[...] = contrib

    @pl.when(k > 0)
    def _():
        cnt_ref[...] += rc
        o_ref[...] += contrib

    @pl.when(k == nk - 1)
    def _():
        o_ref[...] = o_ref[...] / cnt_ref[...]


def _ego_pallas(Bbf, x_pad):
    grid = (NP // EGO_BI, NP // EGO_BK)
    return pl.pallas_call(
        _ego_body,
        grid=grid,
        in_specs=[
            pl.BlockSpec((EGO_BI, NP), lambda i, k: (i, 0)),
            pl.BlockSpec((NP, EGO_BK), lambda i, k: (0, k)),
            pl.BlockSpec((EGO_BK, D), lambda i, k: (k, 0)),
        ],
        out_specs=pl.BlockSpec((EGO_BI, D), lambda i, k: (i, 0)),
        out_shape=jax.ShapeDtypeStruct((NP, D), jnp.float32),
        scratch_shapes=[pltpu.VMEM((EGO_BI, 1), jnp.float32)],
    )(Bbf, Bbf, x_pad)


def _mid_body(x_ref, ego_ref, cutn_ref, cutd_ref, cosn_ref, cosd_ref,
              we_ref, be_ref, wc_ref, bc_ref, wco_ref, bco_ref, wg_ref, bg_ref,
              he_ref, hc_ref, ho_ref, glob_ref):
    x = x_ref[...]
    den = cutd_ref[...]
    has_nb = den > 0.0
    cut = jnp.where(has_nb, cutn_ref[...] / jnp.maximum(den, 1e-12), x)
    cosd = cosd_ref[...]
    cosine = jnp.where(has_nb, cosn_ref[...] / jnp.maximum(cosd, 1e-12), x)
    ego = ego_ref[...]
    he_ref[...] = jnp.dot(ego, we_ref[...].T,
                          preferred_element_type=jnp.float32) + be_ref[...]
    hc_ref[...] = jnp.dot(cut, wc_ref[...].T,
                          preferred_element_type=jnp.float32) + bc_ref[...]
    ho_ref[...] = jnp.dot(cosine, wco_ref[...].T,
                          preferred_element_type=jnp.float32) + bco_ref[...]
    glob_ref[...] = jnp.dot(x, wg_ref[...].T,
                            preferred_element_type=jnp.float32) + bg_ref[...]


def _mid_pallas(x, ego, cut_num, cut_den, cos_num, cos_den,
                W_ego, b_ego, W_cut, b_cut, W_cos, b_cos, W_glob, b_glob):
    BR = 1000
    grid = (N // BR,)
    row = pl.BlockSpec((BR, D), lambda i: (i, 0))
    rowc = pl.BlockSpec((BR, 1), lambda i: (i, 0))
    wspec = pl.BlockSpec((D, D), lambda i: (0, 0))
    bspec = pl.BlockSpec((1, D), lambda i: (0, 0))
    return pl.pallas_call(
        _mid_body,
        grid=grid,
        in_specs=[row, row, row, rowc, row, rowc,
                  wspec, bspec, wspec, bspec, wspec, bspec, wspec, bspec],
        out_specs=[row, row, row, row],
        out_shape=[jax.ShapeDtypeStruct((N, D), jnp.float32)] * 4,
    )(x, ego, cut_num, cut_den, cos_num, cos_den,
      W_ego, b_ego.reshape(1, D), W_cut, b_cut.reshape(1, D),
      W_cos, b_cos.reshape(1, D), W_glob, b_glob.reshape(1, D))


def _tail_pallas(agge, aggc, aggo, glob, W_fc, b_fc):
    BR = 1000
    grid = (N // BR,)
    row = pl.BlockSpec((BR, D), lambda i: (i, 0))
    rowp = pl.BlockSpec((1, BR, D), lambda i: (0, i, 0))

    def body(agge_ref, aggc_ref, aggo_ref, glob_ref, wfc_ref, bfc_ref, o_ref):
        comb = jnp.concatenate([
            jax.nn.relu(agge_ref[...]),
            jax.nn.relu(aggc_ref[...]),
            jax.nn.relu(aggo_ref[0] + aggo_ref[1]),
            glob_ref[...],
        ], axis=1)
        logits = jnp.dot(comb, wfc_ref[...].T,
                         preferred_element_type=jnp.float32) + bfc_ref[...]
        m = jnp.max(logits, axis=1, keepdims=True)
        sh = logits - m
        lse = jnp.log(jnp.sum(jnp.exp(sh), axis=1, keepdims=True))
        o_ref[...] = sh - lse

    return pl.pallas_call(
        body,
        grid=grid,
        in_specs=[
            row, row,
            pl.BlockSpec((NCORE, BR, D), lambda i: (0, i, 0)),
            row,
            pl.BlockSpec((D, 4 * D), lambda i: (0, 0)),
            pl.BlockSpec((1, D), lambda i: (0, 0)),
        ],
        out_specs=row,
        out_shape=jax.ShapeDtypeStruct((N, D), jnp.float32),
    )(agge, aggc, aggo, glob, W_fc, b_fc.reshape(1, D))


def kernel(x, edge_index, W_ego, b_ego, W_cut, b_cut, W_cos, b_cos,
           W_glob, b_glob, W_fc, b_fc):
    src = edge_index[0]
    dst = edge_index[1]

    pad_src = N + (jnp.arange(EPAD - E, dtype=jnp.int32) % 16)
    pad_dst = jnp.full((EPAD - E,), N, jnp.int32)
    src_pad = jnp.concatenate([src, pad_src])
    dst_pad = jnp.concatenate([dst, pad_dst])
    B01 = _adjacency_sc(src_pad, dst_pad).reshape(NP, NP)
    Bbf = B01.astype(jnp.bfloat16)
    x_pad = jnp.pad(x, ((0, NP - N), (0, 0)))

    ego = _ego_pallas(Bbf, x_pad)[:N]

    cflat = _ctab_pallas(x, x_pad).reshape(-1)
    cut_num, cos_num, cut_den, sume = _cutcos_sc(
        src_pad, dst_pad, cflat, x_pad)

    he, hc, ho, glob = _mid_pallas(
        x, ego, cut_num[:N], cut_den[:N].reshape(N, 1),
        cos_num[:N], sume[:N].reshape(N, 1),
        W_ego, b_ego, W_cut, b_cut, W_cos, b_cos, W_glob, b_glob)

    zpad = ((0, NT - N), (0, 0))
    agge, aggc, aggo = _mpagg_sc(
        src_pad, dst_pad,
        jnp.pad(he, zpad), jnp.pad(hc, zpad), jnp.pad(ho, zpad))

    return _tail_pallas(agge[:N], aggc[:N], aggo[:, :N], glob, W_fc, b_fc)


# speedup vs baseline: 7.5574x; 1.2077x over previous
import jax
import jax.numpy as jnp
from jax import lax
from jax.experimental import pallas as pl
from jax.experimental.pallas import tpu as pltpu
from jax.experimental.pallas import tpu_sc as plsc

N = 10000
E = 320000
D = 128
NP = 10240

NCORE = 2
NSUB = 16
CHUNK = 128
CPT = 157
EPAD = NSUB * CPT * CHUNK
NCHUNK = EPAD // CHUNK
ZONE = NP // NCORE

NT = NP
ZPT = NT // NSUB


def _fill(ref, n, val):
    v = jnp.full((16,), val, ref.dtype)

    def body(i, _):
        ref[pl.ds(i * 16, 16)] = v
        return 0

    lax.fori_loop(0, n // 16, body, 0)


def _zero2d(ref, nrow, cols):
    nv = cols // 16

    def body(i, _):
        r = i // nv
        cblk = (i % nv) * 16
        ref[r, pl.ds(cblk, 16)] = jnp.zeros((16,), jnp.float32)
        return 0

    lax.fori_loop(0, nrow * nv, body, 0)


ZBUF = 32768


def _adj_body(src_hbm, dst_hbm, out_hbm, zbuf, srcv, dstv, idxv, onesv):
    c = lax.axis_index("c")
    s = lax.axis_index("s")
    zelem = ZONE * NP // NSUB
    zbase = c * (ZONE * NP) + s * zelem
    _fill(zbuf, ZBUF, 0.0)

    def zcopy(i, _):
        pltpu.sync_copy(zbuf, out_hbm.at[pl.ds(zbase + i * ZBUF, ZBUF)])
        return 0

    lax.fori_loop(0, zelem // ZBUF, zcopy, 0)
    plsc.subcore_barrier()

    _fill(onesv, CHUNK, 1.0)
    zlo = c * ZONE
    zhi = zlo + ZONE

    def chunk_body(j, _):
        off = (s * CPT + j) * CHUNK
        pltpu.sync_copy(src_hbm.at[pl.ds(off, CHUNK)], srcv)
        pltpu.sync_copy(dst_hbm.at[pl.ds(off, CHUNK)], dstv)
        for v in range(CHUNK // 16):
            sl = pl.ds(v * 16, 16)
            sv = srcv[sl]
            dv = dstv[sl]
            fl = dv * NP + sv
            inz = (dv >= zlo) & (dv < zhi)
            trow = zlo + (sv & 4095)
            idxv[sl] = jnp.where(inz, fl, trow * (NP + 1))
        pltpu.sync_copy(onesv, out_hbm.at[idxv])
        return 0

    lax.fori_loop(0, CPT, chunk_body, 0)


def _adjacency_sc(src_pad, dst_pad):
    mesh = plsc.VectorSubcoreMesh(core_axis_name="c", subcore_axis_name="s")
    f = pl.kernel(
        _adj_body,
        out_type=jax.ShapeDtypeStruct((NP * NP,), jnp.float32),
        mesh=mesh,
        scratch_types=[
            pltpu.VMEM((ZBUF,), jnp.float32),
            pltpu.VMEM((CHUNK,), jnp.int32),
            pltpu.VMEM((CHUNK,), jnp.int32),
            pltpu.VMEM((CHUNK,), jnp.int32),
            pltpu.VMEM((CHUNK,), jnp.float32),
        ],
    )
    return f(src_pad, dst_pad)


def _cutcos_body(src_hbm, dst_hbm, cflat_hbm, x_hbm,
                 cutn_hbm, cosn_hbm, den_hbm, sume_hbm,
                 acc_t, acc1_t, buf2d, z1d,
                 srcv, dstv, idxv, ev, onesv, rows):
    c = lax.axis_index("c")
    s = lax.axis_index("s")

    _zero2d(buf2d, CHUNK, D)
    for kk in range(ZPT // CHUNK):
        pltpu.sync_copy(buf2d, acc_t.at[pl.ds(s * ZPT + kk * CHUNK, CHUNK), :])

    @pl.when(s == 0)
    def _():
        _fill(z1d, 2048, 0.0)
        for kk in range(NT // 2048):
            pltpu.sync_copy(z1d, acc1_t.at[pl.ds(kk * 2048, 2048)])

    plsc.subcore_barrier()
    _fill(onesv, CHUNK, 1.0)

    def chunk_cut(j, _):
        off = (s * CPT + j) * CHUNK
        pltpu.sync_copy(src_hbm.at[pl.ds(off, CHUNK)], srcv)
        pltpu.sync_copy(dst_hbm.at[pl.ds(off, CHUNK)], dstv)
        pltpu.sync_copy(x_hbm.at[dstv], rows)
        pltpu.sync_copy(rows, acc_t.at[srcv], add=True)
        pltpu.sync_copy(onesv, acc1_t.at[srcv], add=True)
        return 0

    def chunk_cos(j, _):
        off = (s * CPT + j) * CHUNK
        pltpu.sync_copy(src_hbm.at[pl.ds(off, CHUNK)], srcv)
        pltpu.sync_copy(dst_hbm.at[pl.ds(off, CHUNK)], dstv)
        for v in range(CHUNK // 16):
            sl = pl.ds(v * 16, 16)
            idxv[sl] = srcv[sl] * NP + dstv[sl]
        pltpu.sync_copy(cflat_hbm.at[idxv], ev)
        for v in range(CHUNK // 16):
            sl = pl.ds(v * 16, 16)
            ev[sl] = jnp.exp(ev[sl])
        pltpu.sync_copy(x_hbm.at[dstv], rows)

        def scale(g, _):
            evec = ev[pl.ds(g * 16, 16)]
            for rr in range(16):
                eb = evec.at[jnp.full((16,), rr, jnp.int32)].get(
                    mode="promise_in_bounds")
                r = g * 16 + rr
                for q in range(D // 16):
                    qs = pl.ds(q * 16, 16)
                    rows[r, qs] = rows[r, qs] * eb
            return 0

        lax.fori_loop(0, CHUNK // 16, scale, 0)
        pltpu.sync_copy(rows, acc_t.at[srcv], add=True)
        pltpu.sync_copy(ev, acc1_t.at[srcv], add=True)
        return 0

    @pl.when(c == 0)
    def _():
        lax.fori_loop(0, CPT, chunk_cut, 0)

    @pl.when(c == 1)
    def _():
        lax.fori_loop(0, CPT, chunk_cos, 0)

    plsc.subcore_barrier()

    r0 = s * ZPT
    for kk in range(ZPT // CHUNK):
        rr = pl.ds(r0 + kk * CHUNK, CHUNK)
        pltpu.sync_copy(acc_t.at[rr, :], buf2d)

        @pl.when(c == 0)
        def _():
            pltpu.sync_copy(buf2d, cutn_hbm.at[rr, :])

        @pl.when(c == 1)
        def _():
            pltpu.sync_copy(buf2d, cosn_hbm.at[rr, :])

    @pl.when(s == 0)
    def _():
        for kk in range(NT // 2048):
            r2 = pl.ds(kk * 2048, 2048)
            pltpu.sync_copy(acc1_t.at[r2], z1d)

            @pl.when(c == 0)
            def _():
                pltpu.sync_copy(z1d, den_hbm.at[r2])

            @pl.when(c == 1)
            def _():
                pltpu.sync_copy(z1d, sume_hbm.at[r2])


def _cutcos_sc(src_pad, dst_pad, cflat, x_pad):
    mesh = plsc.VectorSubcoreMesh(core_axis_name="c", subcore_axis_name="s")
    f = pl.kernel(
        _cutcos_body,
        out_type=[
            jax.ShapeDtypeStruct((NT, D), jnp.float32),
            jax.ShapeDtypeStruct((NT, D), jnp.float32),
            jax.ShapeDtypeStruct((NT,), jnp.float32),
            jax.ShapeDtypeStruct((NT,), jnp.float32),
        ],
        mesh=mesh,
        scratch_types=[
            pltpu.VMEM_SHARED((NT, D), jnp.float32),
            pltpu.VMEM_SHARED((NT,), jnp.float32),
            pltpu.VMEM((CHUNK, D), jnp.float32),
            pltpu.VMEM((2048,), jnp.float32),
            pltpu.VMEM((CHUNK,), jnp.int32),
            pltpu.VMEM((CHUNK,), jnp.int32),
            pltpu.VMEM((CHUNK,), jnp.int32),
            pltpu.VMEM((CHUNK,), jnp.float32),
            pltpu.VMEM((CHUNK,), jnp.float32),
            pltpu.VMEM((CHUNK, D), jnp.float32),
        ],
    )
    return f(src_pad, dst_pad, cflat, x_pad)


def _mpagg_body(src_hbm, dst_hbm, he_hbm, hc_hbm, ho_hbm,
                agge_hbm, aggc_hbm, aggo_hbm,
                acc_t, buf2d, srcv, dstv, rows):
    c = lax.axis_index("c")
    s = lax.axis_index("s")

    def zero_acc():
        _zero2d(buf2d, CHUNK, D)
        for kk in range(ZPT // CHUNK):
            pltpu.sync_copy(
                buf2d, acc_t.at[pl.ds(s * ZPT + kk * CHUNK, CHUNK), :])

    def accumulate(h_hbm, off):
        pltpu.sync_copy(src_hbm.at[pl.ds(off, CHUNK)], srcv)
        pltpu.sync_copy(dst_hbm.at[pl.ds(off, CHUNK)], dstv)
        pltpu.sync_copy(h_hbm.at[srcv], rows)
        pltpu.sync_copy(rows, acc_t.at[dstv], add=True)

    def copy_out(dst_hbm_plane):
        r0 = s * ZPT
        for kk in range(ZPT // CHUNK):
            rr = pl.ds(r0 + kk * CHUNK, CHUNK)
            pltpu.sync_copy(acc_t.at[rr, :], buf2d)
            pltpu.sync_copy(buf2d, dst_hbm_plane.at[rr, :])

    zero_acc()
    plsc.subcore_barrier()

    def p1_e(j, _):
        accumulate(he_hbm, (s * CPT + j) * CHUNK)
        return 0

    def p1_c(j, _):
        accumulate(hc_hbm, (s * CPT + j) * CHUNK)
        return 0

    @pl.when(c == 0)
    def _():
        lax.fori_loop(0, CPT, p1_e, 0)

    @pl.when(c == 1)
    def _():
        lax.fori_loop(0, CPT, p1_c, 0)

    plsc.subcore_barrier()

    @pl.when(c == 0)
    def _():
        copy_out(agge_hbm)

    @pl.when(c == 1)
    def _():
        copy_out(aggc_hbm)

    plsc.subcore_barrier()
    zero_acc()
    plsc.subcore_barrier()

    half_chunks = NCHUNK // 2

    def p2(j, _):
        cid = s + j * NSUB

        @pl.when(cid < half_chunks)
        def _():
            accumulate(ho_hbm, (c * half_chunks + cid) * CHUNK)

        return 0

    lax.fori_loop(0, (half_chunks + NSUB - 1) // NSUB, p2, 0)
    plsc.subcore_barrier()
    copy_out(aggo_hbm.at[c])


def _mpagg_sc(src_pad, dst_pad, he, hc, ho):
    mesh = plsc.VectorSubcoreMesh(core_axis_name="c", subcore_axis_name="s")
    f = pl.kernel(
        _mpagg_body,
        out_type=[
            jax.ShapeDtypeStruct((NT, D), jnp.float32),
            jax.ShapeDtypeStruct((NT, D), jnp.float32),
            jax.ShapeDtypeStruct((NCORE, NT, D), jnp.float32),
        ],
        mesh=mesh,
        scratch_types=[
            pltpu.VMEM_SHARED((NT, D), jnp.float32),
            pltpu.VMEM((CHUNK, D), jnp.float32),
            pltpu.VMEM((CHUNK,), jnp.int32),
            pltpu.VMEM((CHUNK,), jnp.int32),
            pltpu.VMEM((CHUNK, D), jnp.float32),
        ],
    )
    return f(src_pad, dst_pad, he, hc, ho)


CB_I = 1000
CB_J = 1024


def _ctab_body(xi_ref, xj_ref, o_ref):
    xi = xi_ref[...]
    ni = xi / jnp.maximum(
        jnp.sqrt(jnp.sum(xi * xi, axis=1, keepdims=True)), 1e-12)
    xj = xj_ref[...]
    nj = xj / jnp.maximum(
        jnp.sqrt(jnp.sum(xj * xj, axis=1, keepdims=True)), 1e-12)
    o_ref[...] = lax.dot_general(ni, nj, (((1,), (1,)), ((), ())),
                                 preferred_element_type=jnp.float32)


def _ctab_pallas(x, x_pad):
    grid = (N // CB_I, NP // CB_J)
    return pl.pallas_call(
        _ctab_body,
        grid=grid,
        in_specs=[
            pl.BlockSpec((CB_I, D), lambda i, j: (i, 0)),
            pl.BlockSpec((CB_J, D), lambda i, j: (j, 0)),
        ],
        out_specs=pl.BlockSpec((CB_I, CB_J), lambda i, j: (i, j)),
        out_shape=jax.ShapeDtypeStruct((N, NP), jnp.float32),
    )(x, x_pad)


EGO_BI = 1024
EGO_BK = 256


def _ego_body(a_ref, b_ref, x_ref, o_ref, cnt_ref):
    k = pl.program_id(1)
    nk = pl.num_programs(1)
    counts = jnp.dot(a_ref[...], b_ref[...], preferred_element_type=jnp.float32)
    direct = a_ref[:, pl.ds(k * EGO_BK, EGO_BK)].astype(jnp.float32)
    tot = counts + direct
    i = pl.program_id(0)
    rows = i * EGO_BI + lax.broadcasted_iota(jnp.int32, (EGO_BI, EGO_BK), 0)
    cols = k * EGO_BK + lax.broadcasted_iota(jnp.int32, (EGO_BI, EGO_BK), 1)
    reach = jnp.where((tot > 0.5) | (rows == cols), 1.0, 0.0)
    contrib = jnp.dot(reach, x_ref[...], preferred_element_type=jnp.float32)
    rc = jnp.sum(reach, axis=1, keepdims=True)

    @pl.when(k == 0)
    def _():
        cnt_ref[...] = rc
        o_ref[...] = contrib

    @pl.when(k > 0)
    def _():
        cnt_ref[...] += rc
        o_ref[...] += contrib

    @pl.when(k == nk - 1)
    def _():
        o_ref[...] = o_ref[...] / cnt_ref[...]


def _ego_pallas(Bbf, x_pad):
    grid = (NP // EGO_BI, NP // EGO_BK)
    return pl.pallas_call(
        _ego_body,
        grid=grid,
        in_specs=[
            pl.BlockSpec((EGO_BI, NP), lambda i, k: (i, 0)),
            pl.BlockSpec((NP, EGO_BK), lambda i, k: (0, k)),
            pl.BlockSpec((EGO_BK, D), lambda i, k: (k, 0)),
        ],
        out_specs=pl.BlockSpec((EGO_BI, D), lambda i, k: (i, 0)),
        out_shape=jax.ShapeDtypeStruct((NP, D), jnp.float32),
        scratch_shapes=[pltpu.VMEM((EGO_BI, 1), jnp.float32)],
    )(Bbf, Bbf, x_pad)


def _mid_body(x_ref, ego_ref, cutn_ref, cutd_ref, cosn_ref, cosd_ref,
              we_ref, be_ref, wc_ref, bc_ref, wco_ref, bco_ref, wg_ref, bg_ref,
              he_ref, hc_ref, ho_ref, glob_ref):
    x = x_ref[...]
    den = cutd_ref[...]
    has_nb = den > 0.0
    cut = jnp.where(has_nb, cutn_ref[...] / jnp.maximum(den, 1e-12), x)
    cosd = cosd_ref[...]
    cosine = jnp.where(has_nb, cosn_ref[...] / jnp.maximum(cosd, 1e-12), x)
    ego = ego_ref[...]
    he_ref[...] = jnp.dot(ego, we_ref[...].T,
                          preferred_element_type=jnp.float32) + be_ref[...]
    hc_ref[...] = jnp.dot(cut, wc_ref[...].T,
                          preferred_element_type=jnp.float32) + bc_ref[...]
    ho_ref[...] = jnp.dot(cosine, wco_ref[...].T,
                          preferred_element_type=jnp.float32) + bco_ref[...]
    glob_ref[...] = jnp.dot(x, wg_ref[...].T,
                            preferred_element_type=jnp.float32) + bg_ref[...]


def _mid_pallas(x, ego, cut_num, cut_den, cos_num, cos_den,
                W_ego, b_ego, W_cut, b_cut, W_cos, b_cos, W_glob, b_glob):
    BR = 1000
    grid = (N // BR,)
    row = pl.BlockSpec((BR, D), lambda i: (i, 0))
    rowc = pl.BlockSpec((BR, 1), lambda i: (i, 0))
    wspec = pl.BlockSpec((D, D), lambda i: (0, 0))
    bspec = pl.BlockSpec((1, D), lambda i: (0, 0))
    return pl.pallas_call(
        _mid_body,
        grid=grid,
        in_specs=[row, row, row, rowc, row, rowc,
                  wspec, bspec, wspec, bspec, wspec, bspec, wspec, bspec],
        out_specs=[row, row, row, row],
        out_shape=[jax.ShapeDtypeStruct((N, D), jnp.float32)] * 4,
    )(x, ego, cut_num, cut_den, cos_num, cos_den,
      W_ego, b_ego.reshape(1, D), W_cut, b_cut.reshape(1, D),
      W_cos, b_cos.reshape(1, D), W_glob, b_glob.reshape(1, D))


def _tail_pallas(agge, aggc, aggo, glob, W_fc, b_fc):
    BR = 1000
    grid = (N // BR,)
    row = pl.BlockSpec((BR, D), lambda i: (i, 0))
    rowp = pl.BlockSpec((1, BR, D), lambda i: (0, i, 0))

    def body(agge_ref, aggc_ref, aggo_ref, glob_ref, wfc_ref, bfc_ref, o_ref):
        comb = jnp.concatenate([
            jax.nn.relu(agge_ref[...]),
            jax.nn.relu(aggc_ref[...]),
            jax.nn.relu(aggo_ref[0] + aggo_ref[1]),
            glob_ref[...],
        ], axis=1)
        logits = jnp.dot(comb, wfc_ref[...].T,
                         preferred_element_type=jnp.float32) + bfc_ref[...]
        m = jnp.max(logits, axis=1, keepdims=True)
        sh = logits - m
        lse = jnp.log(jnp.sum(jnp.exp(sh), axis=1, keepdims=True))
        o_ref[...] = sh - lse

    return pl.pallas_call(
        body,
        grid=grid,
        in_specs=[
            row, row,
            pl.BlockSpec((NCORE, BR, D), lambda i: (0, i, 0)),
            row,
            pl.BlockSpec((D, 4 * D), lambda i: (0, 0)),
            pl.BlockSpec((1, D), lambda i: (0, 0)),
        ],
        out_specs=row,
        out_shape=jax.ShapeDtypeStruct((N, D), jnp.float32),
    )(agge, aggc, aggo, glob, W_fc, b_fc.reshape(1, D))


def kernel(x, edge_index, W_ego, b_ego, W_cut, b_cut, W_cos, b_cos,
           W_glob, b_glob, W_fc, b_fc):
    src = edge_index[0]
    dst = edge_index[1]

    pad_src = N + (jnp.arange(EPAD - E, dtype=jnp.int32) % 16)
    pad_dst = jnp.full((EPAD - E,), N, jnp.int32)
    src_pad = jnp.concatenate([src, pad_src])
    dst_pad = jnp.concatenate([dst, pad_dst])
    B01 = _adjacency_sc(src_pad, dst_pad).reshape(NP, NP)
    Bbf = B01.astype(jnp.float8_e4m3fn)
    x_pad = jnp.pad(x, ((0, NP - N), (0, 0)))

    ego = _ego_pallas(Bbf, x_pad)[:N]

    cflat = _ctab_pallas(x, x_pad).reshape(-1)
    cut_num, cos_num, cut_den, sume = _cutcos_sc(
        src_pad, dst_pad, cflat, x_pad)

    he, hc, ho, glob = _mid_pallas(
        x, ego, cut_num[:N], cut_den[:N].reshape(N, 1),
        cos_num[:N], sume[:N].reshape(N, 1),
        W_ego, b_ego, W_cut, b_cut, W_cos, b_cos, W_glob, b_glob)

    zpad = ((0, NT - N), (0, 0))
    agge, aggc, aggo = _mpagg_sc(
        src_pad, dst_pad,
        jnp.pad(he, zpad), jnp.pad(hc, zpad), jnp.pad(ho, zpad))

    return _tail_pallas(agge[:N], aggc[:N], aggo[:, :N], glob, W_fc, b_fc)


# speedup vs baseline: 7.9084x; 1.0464x over previous
import jax
import jax.numpy as jnp
from jax import lax
from jax.experimental import pallas as pl
from jax.experimental.pallas import tpu as pltpu
from jax.experimental.pallas import tpu_sc as plsc

N = 10000
E = 320000
D = 128
NP = 10240

NCORE = 2
NSUB = 16
CHUNK = 128
CPT = 157
EPAD = NSUB * CPT * CHUNK
NCHUNK = EPAD // CHUNK
ZONE = NP // NCORE

NT = NP
ZPT = NT // NSUB


def _fill(ref, n, val):
    v = jnp.full((16,), val, ref.dtype)

    def body(i, _):
        ref[pl.ds(i * 16, 16)] = v
        return 0

    lax.fori_loop(0, n // 16, body, 0)


def _zero2d(ref, nrow, cols):
    nv = cols // 16

    def body(i, _):
        r = i // nv
        cblk = (i % nv) * 16
        ref[r, pl.ds(cblk, 16)] = jnp.zeros((16,), jnp.float32)
        return 0

    lax.fori_loop(0, nrow * nv, body, 0)


ZBUF = 32768


def _adj_body(src_hbm, dst_hbm, out_hbm, zbuf, srcv, dstv, idxv, onesv):
    c = lax.axis_index("c")
    s = lax.axis_index("s")
    zelem = ZONE * NP // NSUB
    zbase = c * (ZONE * NP) + s * zelem
    _fill(zbuf, ZBUF, 0.0)

    def zcopy(i, _):
        pltpu.sync_copy(zbuf, out_hbm.at[pl.ds(zbase + i * ZBUF, ZBUF)])
        return 0

    lax.fori_loop(0, zelem // ZBUF, zcopy, 0)
    plsc.subcore_barrier()

    _fill(onesv, CHUNK, 1.0)
    zlo = c * ZONE
    zhi = zlo + ZONE

    def chunk_body(j, _):
        off = (s * CPT + j) * CHUNK
        pltpu.sync_copy(src_hbm.at[pl.ds(off, CHUNK)], srcv)
        pltpu.sync_copy(dst_hbm.at[pl.ds(off, CHUNK)], dstv)
        for v in range(CHUNK // 16):
            sl = pl.ds(v * 16, 16)
            sv = srcv[sl]
            dv = dstv[sl]
            fl = dv * NP + sv
            inz = (dv >= zlo) & (dv < zhi)
            trow = zlo + (sv & 4095)
            idxv[sl] = jnp.where(inz, fl, trow * (NP + 1))
        pltpu.sync_copy(onesv, out_hbm.at[idxv])
        return 0

    lax.fori_loop(0, CPT, chunk_body, 0)


def _adjacency_sc(src_pad, dst_pad):
    mesh = plsc.VectorSubcoreMesh(core_axis_name="c", subcore_axis_name="s")
    f = pl.kernel(
        _adj_body,
        out_type=jax.ShapeDtypeStruct((NP * NP,), jnp.float32),
        mesh=mesh,
        scratch_types=[
            pltpu.VMEM((ZBUF,), jnp.float32),
            pltpu.VMEM((CHUNK,), jnp.int32),
            pltpu.VMEM((CHUNK,), jnp.int32),
            pltpu.VMEM((CHUNK,), jnp.int32),
            pltpu.VMEM((CHUNK,), jnp.float32),
        ],
    )
    return f(src_pad, dst_pad)


def _cutcos_body(src_hbm, dst_hbm, cflat_hbm, x_hbm,
                 cutn_hbm, cosn_hbm, den_hbm, sume_hbm,
                 acc_t, acc1_t, buf2d, z1d,
                 srcv, dstv, idxv, ev, onesv, rows):
    c = lax.axis_index("c")
    s = lax.axis_index("s")

    _zero2d(buf2d, CHUNK, D)
    for kk in range(ZPT // CHUNK):
        pltpu.sync_copy(buf2d, acc_t.at[pl.ds(s * ZPT + kk * CHUNK, CHUNK), :])

    @pl.when(s == 0)
    def _():
        _fill(z1d, 2048, 0.0)
        for kk in range(NT // 2048):
            pltpu.sync_copy(z1d, acc1_t.at[pl.ds(kk * 2048, 2048)])

    plsc.subcore_barrier()
    _fill(onesv, CHUNK, 1.0)

    def chunk_cut(j, _):
        off = (s * CPT + j) * CHUNK
        pltpu.sync_copy(src_hbm.at[pl.ds(off, CHUNK)], srcv)
        pltpu.sync_copy(dst_hbm.at[pl.ds(off, CHUNK)], dstv)
        pltpu.sync_copy(x_hbm.at[dstv], rows)
        pltpu.sync_copy(rows, acc_t.at[srcv], add=True)
        pltpu.sync_copy(onesv, acc1_t.at[srcv], add=True)
        return 0

    def chunk_cos(j, _):
        off = (s * CPT + j) * CHUNK
        pltpu.sync_copy(src_hbm.at[pl.ds(off, CHUNK)], srcv)
        pltpu.sync_copy(dst_hbm.at[pl.ds(off, CHUNK)], dstv)
        for v in range(CHUNK // 16):
            sl = pl.ds(v * 16, 16)
            idxv[sl] = srcv[sl] * NP + dstv[sl]
        pltpu.sync_copy(cflat_hbm.at[idxv], ev)
        for v in range(CHUNK // 16):
            sl = pl.ds(v * 16, 16)
            ev[sl] = jnp.exp(ev[sl])
        pltpu.sync_copy(x_hbm.at[dstv], rows)

        def scale(g, _):
            evec = ev[pl.ds(g * 16, 16)]
            for rr in range(16):
                eb = evec.at[jnp.full((16,), rr, jnp.int32)].get(
                    mode="promise_in_bounds")
                r = g * 16 + rr
                for q in range(D // 16):
                    qs = pl.ds(q * 16, 16)
                    rows[r, qs] = rows[r, qs] * eb
            return 0

        lax.fori_loop(0, CHUNK // 16, scale, 0)
        pltpu.sync_copy(rows, acc_t.at[srcv], add=True)
        pltpu.sync_copy(ev, acc1_t.at[srcv], add=True)
        return 0

    @pl.when(c == 0)
    def _():
        lax.fori_loop(0, CPT, chunk_cut, 0)

    @pl.when(c == 1)
    def _():
        lax.fori_loop(0, CPT, chunk_cos, 0)

    plsc.subcore_barrier()

    r0 = s * ZPT
    for kk in range(ZPT // CHUNK):
        rr = pl.ds(r0 + kk * CHUNK, CHUNK)
        pltpu.sync_copy(acc_t.at[rr, :], buf2d)

        @pl.when(c == 0)
        def _():
            pltpu.sync_copy(buf2d, cutn_hbm.at[rr, :])

        @pl.when(c == 1)
        def _():
            pltpu.sync_copy(buf2d, cosn_hbm.at[rr, :])

    @pl.when(s == 0)
    def _():
        for kk in range(NT // 2048):
            r2 = pl.ds(kk * 2048, 2048)
            pltpu.sync_copy(acc1_t.at[r2], z1d)

            @pl.when(c == 0)
            def _():
                pltpu.sync_copy(z1d, den_hbm.at[r2])

            @pl.when(c == 1)
            def _():
                pltpu.sync_copy(z1d, sume_hbm.at[r2])


def _cutcos_sc(src_pad, dst_pad, cflat, x_pad):
    mesh = plsc.VectorSubcoreMesh(core_axis_name="c", subcore_axis_name="s")
    f = pl.kernel(
        _cutcos_body,
        out_type=[
            jax.ShapeDtypeStruct((NT, D), jnp.float32),
            jax.ShapeDtypeStruct((NT, D), jnp.float32),
            jax.ShapeDtypeStruct((NT,), jnp.float32),
            jax.ShapeDtypeStruct((NT,), jnp.float32),
        ],
        mesh=mesh,
        scratch_types=[
            pltpu.VMEM_SHARED((NT, D), jnp.float32),
            pltpu.VMEM_SHARED((NT,), jnp.float32),
            pltpu.VMEM((CHUNK, D), jnp.float32),
            pltpu.VMEM((2048,), jnp.float32),
            pltpu.VMEM((CHUNK,), jnp.int32),
            pltpu.VMEM((CHUNK,), jnp.int32),
            pltpu.VMEM((CHUNK,), jnp.int32),
            pltpu.VMEM((CHUNK,), jnp.float32),
            pltpu.VMEM((CHUNK,), jnp.float32),
            pltpu.VMEM((CHUNK, D), jnp.float32),
        ],
    )
    return f(src_pad, dst_pad, cflat, x_pad)


def _mpagg_body(src_hbm, dst_hbm, he_hbm, hc_hbm, ho_hbm,
                agge_hbm, aggc_hbm, aggo_hbm,
                acc_t, buf2d, srcv, dstv, rows):
    c = lax.axis_index("c")
    s = lax.axis_index("s")

    def zero_acc():
        _zero2d(buf2d, CHUNK, D)
        for kk in range(ZPT // CHUNK):
            pltpu.sync_copy(
                buf2d, acc_t.at[pl.ds(s * ZPT + kk * CHUNK, CHUNK), :])

    def accumulate(h_hbm, off):
        pltpu.sync_copy(src_hbm.at[pl.ds(off, CHUNK)], srcv)
        pltpu.sync_copy(dst_hbm.at[pl.ds(off, CHUNK)], dstv)
        pltpu.sync_copy(h_hbm.at[srcv], rows)
        pltpu.sync_copy(rows, acc_t.at[dstv], add=True)

    def copy_out(dst_hbm_plane):
        r0 = s * ZPT
        for kk in range(ZPT // CHUNK):
            rr = pl.ds(r0 + kk * CHUNK, CHUNK)
            pltpu.sync_copy(acc_t.at[rr, :], buf2d)
            pltpu.sync_copy(buf2d, dst_hbm_plane.at[rr, :])

    zero_acc()
    plsc.subcore_barrier()

    def p1_e(j, _):
        accumulate(he_hbm, (s * CPT + j) * CHUNK)
        return 0

    def p1_c(j, _):
        accumulate(hc_hbm, (s * CPT + j) * CHUNK)
        return 0

    @pl.when(c == 0)
    def _():
        lax.fori_loop(0, CPT, p1_e, 0)

    @pl.when(c == 1)
    def _():
        lax.fori_loop(0, CPT, p1_c, 0)

    plsc.subcore_barrier()

    @pl.when(c == 0)
    def _():
        copy_out(agge_hbm)

    @pl.when(c == 1)
    def _():
        copy_out(aggc_hbm)

    plsc.subcore_barrier()
    zero_acc()
    plsc.subcore_barrier()

    half_chunks = NCHUNK // 2

    def p2(j, _):
        cid = s + j * NSUB

        @pl.when(cid < half_chunks)
        def _():
            accumulate(ho_hbm, (c * half_chunks + cid) * CHUNK)

        return 0

    lax.fori_loop(0, (half_chunks + NSUB - 1) // NSUB, p2, 0)
    plsc.subcore_barrier()
    copy_out(aggo_hbm.at[c])


def _mpagg_sc(src_pad, dst_pad, he, hc, ho):
    mesh = plsc.VectorSubcoreMesh(core_axis_name="c", subcore_axis_name="s")
    f = pl.kernel(
        _mpagg_body,
        out_type=[
            jax.ShapeDtypeStruct((NT, D), jnp.float32),
            jax.ShapeDtypeStruct((NT, D), jnp.float32),
            jax.ShapeDtypeStruct((NCORE, NT, D), jnp.float32),
        ],
        mesh=mesh,
        scratch_types=[
            pltpu.VMEM_SHARED((NT, D), jnp.float32),
            pltpu.VMEM((CHUNK, D), jnp.float32),
            pltpu.VMEM((CHUNK,), jnp.int32),
            pltpu.VMEM((CHUNK,), jnp.int32),
            pltpu.VMEM((CHUNK, D), jnp.float32),
        ],
    )
    return f(src_pad, dst_pad, he, hc, ho)


CB_I = 1000
CB_J = 1024


def _ctab_body(xi_ref, xj_ref, o_ref):
    xi = xi_ref[...]
    ni = xi / jnp.maximum(
        jnp.sqrt(jnp.sum(xi * xi, axis=1, keepdims=True)), 1e-12)
    xj = xj_ref[...]
    nj = xj / jnp.maximum(
        jnp.sqrt(jnp.sum(xj * xj, axis=1, keepdims=True)), 1e-12)
    o_ref[...] = lax.dot_general(ni, nj, (((1,), (1,)), ((), ())),
                                 preferred_element_type=jnp.float32)


def _ctab_pallas(x, x_pad):
    grid = (N // CB_I, NP // CB_J)
    return pl.pallas_call(
        _ctab_body,
        grid=grid,
        in_specs=[
            pl.BlockSpec((CB_I, D), lambda i, j: (i, 0)),
            pl.BlockSpec((CB_J, D), lambda i, j: (j, 0)),
        ],
        out_specs=pl.BlockSpec((CB_I, CB_J), lambda i, j: (i, j)),
        out_shape=jax.ShapeDtypeStruct((N, NP), jnp.float32),
    )(x, x_pad)


EGO_BI = 1024
EGO_BK = 512


def _ego_body(a_ref, b_ref, x_ref, o_ref, cnt_ref):
    k = pl.program_id(1)
    nk = pl.num_programs(1)
    counts = jnp.dot(a_ref[...], b_ref[...], preferred_element_type=jnp.float32)
    direct = a_ref[:, pl.ds(k * EGO_BK, EGO_BK)].astype(jnp.float32)
    tot = counts + direct
    i = pl.program_id(0)
    rows = i * EGO_BI + lax.broadcasted_iota(jnp.int32, (EGO_BI, EGO_BK), 0)
    cols = k * EGO_BK + lax.broadcasted_iota(jnp.int32, (EGO_BI, EGO_BK), 1)
    reach = jnp.where((tot > 0.5) | (rows == cols), 1.0, 0.0)
    contrib = jnp.dot(reach, x_ref[...], preferred_element_type=jnp.float32)
    rc = jnp.sum(reach, axis=1, keepdims=True)

    @pl.when(k == 0)
    def _():
        cnt_ref[...] = rc
        o_ref[...] = contrib

    @pl.when(k > 0)
    def _():
        cnt_ref[...] += rc
        o_ref[...] += contrib

    @pl.when(k == nk - 1)
    def _():
        o_ref[...] = o_ref[...] / cnt_ref[...]


def _ego_pallas(Bbf, x_pad):
    grid = (NP // EGO_BI, NP // EGO_BK)
    return pl.pallas_call(
        _ego_body,
        grid=grid,
        in_specs=[
            pl.BlockSpec((EGO_BI, NP), lambda i, k: (i, 0)),
            pl.BlockSpec((NP, EGO_BK), lambda i, k: (0, k)),
            pl.BlockSpec((EGO_BK, D), lambda i, k: (k, 0)),
        ],
        out_specs=pl.BlockSpec((EGO_BI, D), lambda i, k: (i, 0)),
        out_shape=jax.ShapeDtypeStruct((NP, D), jnp.float32),
        scratch_shapes=[pltpu.VMEM((EGO_BI, 1), jnp.float32)],
    )(Bbf, Bbf, x_pad)


def _mid_body(x_ref, ego_ref, cutn_ref, cutd_ref, cosn_ref, cosd_ref,
              we_ref, be_ref, wc_ref, bc_ref, wco_ref, bco_ref, wg_ref, bg_ref,
              he_ref, hc_ref, ho_ref, glob_ref):
    x = x_ref[...]
    den = cutd_ref[...]
    has_nb = den > 0.0
    cut = jnp.where(has_nb, cutn_ref[...] / jnp.maximum(den, 1e-12), x)
    cosd = cosd_ref[...]
    cosine = jnp.where(has_nb, cosn_ref[...] / jnp.maximum(cosd, 1e-12), x)
    ego = ego_ref[...]
    he_ref[...] = jnp.dot(ego, we_ref[...].T,
                          preferred_element_type=jnp.float32) + be_ref[...]
    hc_ref[...] = jnp.dot(cut, wc_ref[...].T,
                          preferred_element_type=jnp.float32) + bc_ref[...]
    ho_ref[...] = jnp.dot(cosine, wco_ref[...].T,
                          preferred_element_type=jnp.float32) + bco_ref[...]
    glob_ref[...] = jnp.dot(x, wg_ref[...].T,
                            preferred_element_type=jnp.float32) + bg_ref[...]


def _mid_pallas(x, ego, cut_num, cut_den, cos_num, cos_den,
                W_ego, b_ego, W_cut, b_cut, W_cos, b_cos, W_glob, b_glob):
    BR = 1000
    grid = (N // BR,)
    row = pl.BlockSpec((BR, D), lambda i: (i, 0))
    rowc = pl.BlockSpec((BR, 1), lambda i: (i, 0))
    wspec = pl.BlockSpec((D, D), lambda i: (0, 0))
    bspec = pl.BlockSpec((1, D), lambda i: (0, 0))
    return pl.pallas_call(
        _mid_body,
        grid=grid,
        in_specs=[row, row, row, rowc, row, rowc,
                  wspec, bspec, wspec, bspec, wspec, bspec, wspec, bspec],
        out_specs=[row, row, row, row],
        out_shape=[jax.ShapeDtypeStruct((N, D), jnp.float32)] * 4,
    )(x, ego, cut_num, cut_den, cos_num, cos_den,
      W_ego, b_ego.reshape(1, D), W_cut, b_cut.reshape(1, D),
      W_cos, b_cos.reshape(1, D), W_glob, b_glob.reshape(1, D))


def _tail_pallas(agge, aggc, aggo, glob, W_fc, b_fc):
    BR = 1000
    grid = (N // BR,)
    row = pl.BlockSpec((BR, D), lambda i: (i, 0))
    rowp = pl.BlockSpec((1, BR, D), lambda i: (0, i, 0))

    def body(agge_ref, aggc_ref, aggo_ref, glob_ref, wfc_ref, bfc_ref, o_ref):
        comb = jnp.concatenate([
            jax.nn.relu(agge_ref[...]),
            jax.nn.relu(aggc_ref[...]),
            jax.nn.relu(aggo_ref[0] + aggo_ref[1]),
            glob_ref[...],
        ], axis=1)
        logits = jnp.dot(comb, wfc_ref[...].T,
                         preferred_element_type=jnp.float32) + bfc_ref[...]
        m = jnp.max(logits, axis=1, keepdims=True)
        sh = logits - m
        lse = jnp.log(jnp.sum(jnp.exp(sh), axis=1, keepdims=True))
        o_ref[...] = sh - lse

    return pl.pallas_call(
        body,
        grid=grid,
        in_specs=[
            row, row,
            pl.BlockSpec((NCORE, BR, D), lambda i: (0, i, 0)),
            row,
            pl.BlockSpec((D, 4 * D), lambda i: (0, 0)),
            pl.BlockSpec((1, D), lambda i: (0, 0)),
        ],
        out_specs=row,
        out_shape=jax.ShapeDtypeStruct((N, D), jnp.float32),
    )(agge, aggc, aggo, glob, W_fc, b_fc.reshape(1, D))


def kernel(x, edge_index, W_ego, b_ego, W_cut, b_cut, W_cos, b_cos,
           W_glob, b_glob, W_fc, b_fc):
    src = edge_index[0]
    dst = edge_index[1]

    pad_src = N + (jnp.arange(EPAD - E, dtype=jnp.int32) % 16)
    pad_dst = jnp.full((EPAD - E,), N, jnp.int32)
    src_pad = jnp.concatenate([src, pad_src])
    dst_pad = jnp.concatenate([dst, pad_dst])
    B01 = _adjacency_sc(src_pad, dst_pad).reshape(NP, NP)
    Bbf = B01.astype(jnp.float8_e4m3fn)
    x_pad = jnp.pad(x, ((0, NP - N), (0, 0)))

    ego = _ego_pallas(Bbf, x_pad)[:N]

    cflat = _ctab_pallas(x, x_pad).reshape(-1)
    cut_num, cos_num, cut_den, sume = _cutcos_sc(
        src_pad, dst_pad, cflat, x_pad)

    he, hc, ho, glob = _mid_pallas(
        x, ego, cut_num[:N], cut_den[:N].reshape(N, 1),
        cos_num[:N], sume[:N].reshape(N, 1),
        W_ego, b_ego, W_cut, b_cut, W_cos, b_cos, W_glob, b_glob)

    zpad = ((0, NT - N), (0, 0))
    agge, aggc, aggo = _mpagg_sc(
        src_pad, dst_pad,
        jnp.pad(he, zpad), jnp.pad(hc, zpad), jnp.pad(ho, zpad))

    return _tail_pallas(agge[:N], aggc[:N], aggo[:, :N], glob, W_fc, b_fc)


# speedup vs baseline: 7.9551x; 1.0059x over previous
import jax
import jax.numpy as jnp
from jax import lax
from jax.experimental import pallas as pl
from jax.experimental.pallas import tpu as pltpu
from jax.experimental.pallas import tpu_sc as plsc

N = 10000
E = 320000
D = 128
NP = 10240

NCORE = 2
NSUB = 16
CHUNK = 128
CPT = 157
EPAD = NSUB * CPT * CHUNK
NCHUNK = EPAD // CHUNK
ZONE = NP // NCORE

NT = NP
ZPT = NT // NSUB


def _fill(ref, n, val):
    v = jnp.full((16,), val, ref.dtype)

    def body(i, _):
        ref[pl.ds(i * 16, 16)] = v
        return 0

    lax.fori_loop(0, n // 16, body, 0)


def _zero2d(ref, nrow, cols):
    nv = cols // 16

    def body(i, _):
        r = i // nv
        cblk = (i % nv) * 16
        ref[r, pl.ds(cblk, 16)] = jnp.zeros((16,), jnp.float32)
        return 0

    lax.fori_loop(0, nrow * nv, body, 0)


ZBUF = 32768


def _adj_body(src_hbm, dst_hbm, out_hbm, zbuf, srcv, dstv, idxv, onesv):
    c = lax.axis_index("c")
    s = lax.axis_index("s")
    zelem = ZONE * NP // NSUB
    zbase = c * (ZONE * NP) + s * zelem
    _fill(zbuf, ZBUF, 0.0)

    def zcopy(i, _):
        pltpu.sync_copy(zbuf, out_hbm.at[pl.ds(zbase + i * ZBUF, ZBUF)])
        return 0

    lax.fori_loop(0, zelem // ZBUF, zcopy, 0)
    plsc.subcore_barrier()

    _fill(onesv, CHUNK, 1.0)
    zlo = c * ZONE
    zhi = zlo + ZONE

    def chunk_body(j, _):
        off = (s * CPT + j) * CHUNK
        pltpu.sync_copy(src_hbm.at[pl.ds(off, CHUNK)], srcv)
        pltpu.sync_copy(dst_hbm.at[pl.ds(off, CHUNK)], dstv)
        for v in range(CHUNK // 16):
            sl = pl.ds(v * 16, 16)
            sv = srcv[sl]
            dv = dstv[sl]
            fl = dv * NP + sv
            inz = (dv >= zlo) & (dv < zhi)
            trow = zlo + (sv & 4095)
            idxv[sl] = jnp.where(inz, fl, trow * (NP + 1))
        pltpu.sync_copy(onesv, out_hbm.at[idxv])
        return 0

    lax.fori_loop(0, CPT, chunk_body, 0)


def _adjacency_sc(src_pad, dst_pad):
    mesh = plsc.VectorSubcoreMesh(core_axis_name="c", subcore_axis_name="s")
    f = pl.kernel(
        _adj_body,
        out_type=jax.ShapeDtypeStruct((NP * NP,), jnp.float32),
        mesh=mesh,
        scratch_types=[
            pltpu.VMEM((ZBUF,), jnp.float32),
            pltpu.VMEM((CHUNK,), jnp.int32),
            pltpu.VMEM((CHUNK,), jnp.int32),
            pltpu.VMEM((CHUNK,), jnp.int32),
            pltpu.VMEM((CHUNK,), jnp.float32),
        ],
    )
    return f(src_pad, dst_pad)


def _cutcos_body(src_hbm, dst_hbm, cflat_hbm, x_hbm,
                 cutn_hbm, cosn_hbm, den_hbm, sume_hbm,
                 acc_t, acc1_t, buf2d, z1d,
                 srcv, dstv, idxv, ev, onesv, rows):
    c = lax.axis_index("c")
    s = lax.axis_index("s")

    _zero2d(buf2d, CHUNK, D)
    for kk in range(ZPT // CHUNK):
        pltpu.sync_copy(buf2d, acc_t.at[pl.ds(s * ZPT + kk * CHUNK, CHUNK), :])

    @pl.when(s == 0)
    def _():
        _fill(z1d, 2048, 0.0)
        for kk in range(NT // 2048):
            pltpu.sync_copy(z1d, acc1_t.at[pl.ds(kk * 2048, 2048)])

    plsc.subcore_barrier()
    _fill(onesv, CHUNK, 1.0)

    def chunk_cut(j, _):
        off = (s * CPT + j) * CHUNK
        pltpu.sync_copy(src_hbm.at[pl.ds(off, CHUNK)], srcv)
        pltpu.sync_copy(dst_hbm.at[pl.ds(off, CHUNK)], dstv)
        pltpu.sync_copy(x_hbm.at[dstv], rows)
        pltpu.sync_copy(rows, acc_t.at[srcv], add=True)
        pltpu.sync_copy(onesv, acc1_t.at[srcv], add=True)
        return 0

    def chunk_cos(j, _):
        off = (s * CPT + j) * CHUNK
        pltpu.sync_copy(src_hbm.at[pl.ds(off, CHUNK)], srcv)
        pltpu.sync_copy(dst_hbm.at[pl.ds(off, CHUNK)], dstv)
        for v in range(CHUNK // 16):
            sl = pl.ds(v * 16, 16)
            idxv[sl] = srcv[sl] * NP + dstv[sl]
        pltpu.sync_copy(cflat_hbm.at[idxv], ev)
        for v in range(CHUNK // 16):
            sl = pl.ds(v * 16, 16)
            ev[sl] = jnp.exp(ev[sl])
        pltpu.sync_copy(x_hbm.at[dstv], rows)

        def scale(g, _):
            evec = ev[pl.ds(g * 16, 16)]
            for rr in range(16):
                eb = evec.at[jnp.full((16,), rr, jnp.int32)].get(
                    mode="promise_in_bounds")
                r = g * 16 + rr
                for q in range(D // 16):
                    qs = pl.ds(q * 16, 16)
                    rows[r, qs] = rows[r, qs] * eb
            return 0

        lax.fori_loop(0, CHUNK // 16, scale, 0)
        pltpu.sync_copy(rows, acc_t.at[srcv], add=True)
        pltpu.sync_copy(ev, acc1_t.at[srcv], add=True)
        return 0

    @pl.when(c == 0)
    def _():
        lax.fori_loop(0, CPT, chunk_cut, 0)

    @pl.when(c == 1)
    def _():
        lax.fori_loop(0, CPT, chunk_cos, 0)

    plsc.subcore_barrier()

    r0 = s * ZPT
    for kk in range(ZPT // CHUNK):
        rr = pl.ds(r0 + kk * CHUNK, CHUNK)
        pltpu.sync_copy(acc_t.at[rr, :], buf2d)

        @pl.when(c == 0)
        def _():
            pltpu.sync_copy(buf2d, cutn_hbm.at[rr, :])

        @pl.when(c == 1)
        def _():
            pltpu.sync_copy(buf2d, cosn_hbm.at[rr, :])

    @pl.when(s == 0)
    def _():
        for kk in range(NT // 2048):
            r2 = pl.ds(kk * 2048, 2048)
            pltpu.sync_copy(acc1_t.at[r2], z1d)

            @pl.when(c == 0)
            def _():
                pltpu.sync_copy(z1d, den_hbm.at[r2])

            @pl.when(c == 1)
            def _():
                pltpu.sync_copy(z1d, sume_hbm.at[r2])


def _cutcos_sc(src_pad, dst_pad, cflat, x_pad):
    mesh = plsc.VectorSubcoreMesh(core_axis_name="c", subcore_axis_name="s")
    f = pl.kernel(
        _cutcos_body,
        out_type=[
            jax.ShapeDtypeStruct((NT, D), jnp.float32),
            jax.ShapeDtypeStruct((NT, D), jnp.float32),
            jax.ShapeDtypeStruct((NT,), jnp.float32),
            jax.ShapeDtypeStruct((NT,), jnp.float32),
        ],
        mesh=mesh,
        scratch_types=[
            pltpu.VMEM_SHARED((NT, D), jnp.float32),
            pltpu.VMEM_SHARED((NT,), jnp.float32),
            pltpu.VMEM((CHUNK, D), jnp.float32),
            pltpu.VMEM((2048,), jnp.float32),
            pltpu.VMEM((CHUNK,), jnp.int32),
            pltpu.VMEM((CHUNK,), jnp.int32),
            pltpu.VMEM((CHUNK,), jnp.int32),
            pltpu.VMEM((CHUNK,), jnp.float32),
            pltpu.VMEM((CHUNK,), jnp.float32),
            pltpu.VMEM((CHUNK, D), jnp.float32),
        ],
    )
    return f(src_pad, dst_pad, cflat, x_pad)


def _mpagg_body(src_hbm, dst_hbm, he_hbm, hc_hbm, ho_hbm,
                agge_hbm, aggc_hbm, aggo_hbm,
                acc_t, buf2d, srcv, dstv, rows):
    c = lax.axis_index("c")
    s = lax.axis_index("s")

    def zero_acc():
        _zero2d(buf2d, CHUNK, D)
        for kk in range(ZPT // CHUNK):
            pltpu.sync_copy(
                buf2d, acc_t.at[pl.ds(s * ZPT + kk * CHUNK, CHUNK), :])

    def accumulate(h_hbm, off):
        pltpu.sync_copy(src_hbm.at[pl.ds(off, CHUNK)], srcv)
        pltpu.sync_copy(dst_hbm.at[pl.ds(off, CHUNK)], dstv)
        pltpu.sync_copy(h_hbm.at[srcv], rows)
        pltpu.sync_copy(rows, acc_t.at[dstv], add=True)

    def copy_out(dst_hbm_plane):
        r0 = s * ZPT
        for kk in range(ZPT // CHUNK):
            rr = pl.ds(r0 + kk * CHUNK, CHUNK)
            pltpu.sync_copy(acc_t.at[rr, :], buf2d)
            pltpu.sync_copy(buf2d, dst_hbm_plane.at[rr, :])

    zero_acc()
    plsc.subcore_barrier()

    def p1_e(j, _):
        accumulate(he_hbm, (s * CPT + j) * CHUNK)
        return 0

    def p1_c(j, _):
        accumulate(hc_hbm, (s * CPT + j) * CHUNK)
        return 0

    @pl.when(c == 0)
    def _():
        lax.fori_loop(0, CPT, p1_e, 0)

    @pl.when(c == 1)
    def _():
        lax.fori_loop(0, CPT, p1_c, 0)

    plsc.subcore_barrier()

    @pl.when(c == 0)
    def _():
        copy_out(agge_hbm)

    @pl.when(c == 1)
    def _():
        copy_out(aggc_hbm)

    plsc.subcore_barrier()
    zero_acc()
    plsc.subcore_barrier()

    half_chunks = NCHUNK // 2

    def p2(j, _):
        cid = s + j * NSUB

        @pl.when(cid < half_chunks)
        def _():
            accumulate(ho_hbm, (c * half_chunks + cid) * CHUNK)

        return 0

    lax.fori_loop(0, (half_chunks + NSUB - 1) // NSUB, p2, 0)
    plsc.subcore_barrier()
    copy_out(aggo_hbm.at[c])


def _mpagg_sc(src_pad, dst_pad, he, hc, ho):
    mesh = plsc.VectorSubcoreMesh(core_axis_name="c", subcore_axis_name="s")
    f = pl.kernel(
        _mpagg_body,
        out_type=[
            jax.ShapeDtypeStruct((NT, D), jnp.float32),
            jax.ShapeDtypeStruct((NT, D), jnp.float32),
            jax.ShapeDtypeStruct((NCORE, NT, D), jnp.float32),
        ],
        mesh=mesh,
        scratch_types=[
            pltpu.VMEM_SHARED((NT, D), jnp.float32),
            pltpu.VMEM((CHUNK, D), jnp.float32),
            pltpu.VMEM((CHUNK,), jnp.int32),
            pltpu.VMEM((CHUNK,), jnp.int32),
            pltpu.VMEM((CHUNK, D), jnp.float32),
        ],
    )
    return f(src_pad, dst_pad, he, hc, ho)


CB_I = 1000
CB_J = 1024


def _ctab_body(xi_ref, xj_ref, o_ref):
    xi = xi_ref[...]
    ni = xi / jnp.maximum(
        jnp.sqrt(jnp.sum(xi * xi, axis=1, keepdims=True)), 1e-12)
    xj = xj_ref[...]
    nj = xj / jnp.maximum(
        jnp.sqrt(jnp.sum(xj * xj, axis=1, keepdims=True)), 1e-12)
    o_ref[...] = lax.dot_general(ni, nj, (((1,), (1,)), ((), ())),
                                 preferred_element_type=jnp.float32)


def _ctab_pallas(x, x_pad):
    grid = (N // CB_I, NP // CB_J)
    return pl.pallas_call(
        _ctab_body,
        grid=grid,
        in_specs=[
            pl.BlockSpec((CB_I, D), lambda i, j: (i, 0)),
            pl.BlockSpec((CB_J, D), lambda i, j: (j, 0)),
        ],
        out_specs=pl.BlockSpec((CB_I, CB_J), lambda i, j: (i, j)),
        out_shape=jax.ShapeDtypeStruct((N, NP), jnp.float32),
    )(x, x_pad)


EGO_BI = 1024
EGO_BK = 1024


def _ego_body(a_ref, b_ref, x_ref, o_ref, cnt_ref):
    k = pl.program_id(1)
    nk = pl.num_programs(1)
    counts = jnp.dot(a_ref[...], b_ref[...], preferred_element_type=jnp.float32)
    direct = a_ref[:, pl.ds(k * EGO_BK, EGO_BK)].astype(jnp.float32)
    tot = counts + direct
    i = pl.program_id(0)
    rows = i * EGO_BI + lax.broadcasted_iota(jnp.int32, (EGO_BI, EGO_BK), 0)
    cols = k * EGO_BK + lax.broadcasted_iota(jnp.int32, (EGO_BI, EGO_BK), 1)
    reach = jnp.where((tot > 0.5) | (rows == cols), 1.0, 0.0)
    contrib = jnp.dot(reach, x_ref[...], preferred_element_type=jnp.float32)
    rc = jnp.sum(reach, axis=1, keepdims=True)

    @pl.when(k == 0)
    def _():
        cnt_ref[...] = rc
        o_ref[...] = contrib

    @pl.when(k > 0)
    def _():
        cnt_ref[...] += rc
        o_ref[...] += contrib

    @pl.when(k == nk - 1)
    def _():
        o_ref[...] = o_ref[...] / cnt_ref[...]


def _ego_pallas(Bbf, x_pad):
    grid = (NP // EGO_BI, NP // EGO_BK)
    return pl.pallas_call(
        _ego_body,
        grid=grid,
        in_specs=[
            pl.BlockSpec((EGO_BI, NP), lambda i, k: (i, 0)),
            pl.BlockSpec((NP, EGO_BK), lambda i, k: (0, k)),
            pl.BlockSpec((EGO_BK, D), lambda i, k: (k, 0)),
        ],
        out_specs=pl.BlockSpec((EGO_BI, D), lambda i, k: (i, 0)),
        out_shape=jax.ShapeDtypeStruct((NP, D), jnp.float32),
        scratch_shapes=[pltpu.VMEM((EGO_BI, 1), jnp.float32)],
    )(Bbf, Bbf, x_pad)


def _mid_body(x_ref, ego_ref, cutn_ref, cutd_ref, cosn_ref, cosd_ref,
              we_ref, be_ref, wc_ref, bc_ref, wco_ref, bco_ref, wg_ref, bg_ref,
              he_ref, hc_ref, ho_ref, glob_ref):
    x = x_ref[...]
    den = cutd_ref[...]
    has_nb = den > 0.0
    cut = jnp.where(has_nb, cutn_ref[...] / jnp.maximum(den, 1e-12), x)
    cosd = cosd_ref[...]
    cosine = jnp.where(has_nb, cosn_ref[...] / jnp.maximum(cosd, 1e-12), x)
    ego = ego_ref[...]
    he_ref[...] = jnp.dot(ego, we_ref[...].T,
                          preferred_element_type=jnp.float32) + be_ref[...]
    hc_ref[...] = jnp.dot(cut, wc_ref[...].T,
                          preferred_element_type=jnp.float32) + bc_ref[...]
    ho_ref[...] = jnp.dot(cosine, wco_ref[...].T,
                          preferred_element_type=jnp.float32) + bco_ref[...]
    glob_ref[...] = jnp.dot(x, wg_ref[...].T,
                            preferred_element_type=jnp.float32) + bg_ref[...]


def _mid_pallas(x, ego, cut_num, cut_den, cos_num, cos_den,
                W_ego, b_ego, W_cut, b_cut, W_cos, b_cos, W_glob, b_glob):
    BR = 1000
    grid = (N // BR,)
    row = pl.BlockSpec((BR, D), lambda i: (i, 0))
    rowc = pl.BlockSpec((BR, 1), lambda i: (i, 0))
    wspec = pl.BlockSpec((D, D), lambda i: (0, 0))
    bspec = pl.BlockSpec((1, D), lambda i: (0, 0))
    return pl.pallas_call(
        _mid_body,
        grid=grid,
        in_specs=[row, row, row, rowc, row, rowc,
                  wspec, bspec, wspec, bspec, wspec, bspec, wspec, bspec],
        out_specs=[row, row, row, row],
        out_shape=[jax.ShapeDtypeStruct((N, D), jnp.float32)] * 4,
    )(x, ego, cut_num, cut_den, cos_num, cos_den,
      W_ego, b_ego.reshape(1, D), W_cut, b_cut.reshape(1, D),
      W_cos, b_cos.reshape(1, D), W_glob, b_glob.reshape(1, D))


def _tail_pallas(agge, aggc, aggo, glob, W_fc, b_fc):
    BR = 1000
    grid = (N // BR,)
    row = pl.BlockSpec((BR, D), lambda i: (i, 0))
    rowp = pl.BlockSpec((1, BR, D), lambda i: (0, i, 0))

    def body(agge_ref, aggc_ref, aggo_ref, glob_ref, wfc_ref, bfc_ref, o_ref):
        comb = jnp.concatenate([
            jax.nn.relu(agge_ref[...]),
            jax.nn.relu(aggc_ref[...]),
            jax.nn.relu(aggo_ref[0] + aggo_ref[1]),
            glob_ref[...],
        ], axis=1)
        logits = jnp.dot(comb, wfc_ref[...].T,
                         preferred_element_type=jnp.float32) + bfc_ref[...]
        m = jnp.max(logits, axis=1, keepdims=True)
        sh = logits - m
        lse = jnp.log(jnp.sum(jnp.exp(sh), axis=1, keepdims=True))
        o_ref[...] = sh - lse

    return pl.pallas_call(
        body,
        grid=grid,
        in_specs=[
            row, row,
            pl.BlockSpec((NCORE, BR, D), lambda i: (0, i, 0)),
            row,
            pl.BlockSpec((D, 4 * D), lambda i: (0, 0)),
            pl.BlockSpec((1, D), lambda i: (0, 0)),
        ],
        out_specs=row,
        out_shape=jax.ShapeDtypeStruct((N, D), jnp.float32),
    )(agge, aggc, aggo, glob, W_fc, b_fc.reshape(1, D))


def kernel(x, edge_index, W_ego, b_ego, W_cut, b_cut, W_cos, b_cos,
           W_glob, b_glob, W_fc, b_fc):
    src = edge_index[0]
    dst = edge_index[1]

    pad_src = N + (jnp.arange(EPAD - E, dtype=jnp.int32) % 16)
    pad_dst = jnp.full((EPAD - E,), N, jnp.int32)
    src_pad = jnp.concatenate([src, pad_src])
    dst_pad = jnp.concatenate([dst, pad_dst])
    B01 = _adjacency_sc(src_pad, dst_pad).reshape(NP, NP)
    Bbf = B01.astype(jnp.float8_e4m3fn)
    x_pad = jnp.pad(x, ((0, NP - N), (0, 0)))

    ego = _ego_pallas(Bbf, x_pad)[:N]

    cflat = _ctab_pallas(x, x_pad).reshape(-1)
    cut_num, cos_num, cut_den, sume = _cutcos_sc(
        src_pad, dst_pad, cflat, x_pad)

    he, hc, ho, glob = _mid_pallas(
        x, ego, cut_num[:N], cut_den[:N].reshape(N, 1),
        cos_num[:N], sume[:N].reshape(N, 1),
        W_ego, b_ego, W_cut, b_cut, W_cos, b_cos, W_glob, b_glob)

    zpad = ((0, NT - N), (0, 0))
    agge, aggc, aggo = _mpagg_sc(
        src_pad, dst_pad,
        jnp.pad(he, zpad), jnp.pad(hc, zpad), jnp.pad(ho, zpad))

    return _tail_pallas(agge[:N], aggc[:N], aggo[:, :N], glob, W_fc, b_fc)
